# baseline (speedup 1.0000x reference)
import functools

import jax
import jax.numpy as jnp
from jax import lax
from jax.experimental import pallas as pl
from jax.experimental.pallas import tpu as pltpu

H = 128
EPS = 1e-5


def _pick_block(m):
    for b in (2000, 1000, 500, 250, 125, 100, 40, 8):
        if m % b == 0:
            return b
    return m


def _bf(x):
    return x


def _k1_body(groups, nparts, *refs):
    part_refs = refs[:nparts]
    w_ref = refs[nparts]
    u_ref, s_ref = refs[nparts + 1], refs[nparts + 2]
    acc = None
    for gi, members in enumerate(groups):
        xg = None
        for pix in members:
            r = part_refs[pix]
            if len(r.shape) == 3:
                xv = r[0] + r[1]
            else:
                xv = r[...]
            xg = xv if xg is None else xg + xv
        wg = w_ref[pl.ds(gi * H, H), :]
        p = jax.lax.dot_general(_bf(xg), _bf(wg), (((1,), (0,)), ((), ())),
                                preferred_element_type=jnp.float32,
                                precision=jax.lax.Precision.HIGHEST)
        acc = p if acc is None else acc + p
    u_ref[...] = acc
    s1 = jnp.sum(acc, axis=0, keepdims=True)
    s2 = jnp.sum(acc * acc, axis=0, keepdims=True)
    st = jnp.concatenate([s1, s2], axis=0)

    @pl.when(pl.program_id(0) == 0)
    def _():
        s_ref[...] = st

    @pl.when(pl.program_id(0) != 0)
    def _():
        s_ref[...] = s_ref[...] + st


def matmul_stats(parts, w1, groups):
    m = parts[0].shape[-2]
    b = _pick_block(m)
    dout = w1.shape[1]
    in_specs = []
    for p in parts:
        if p.ndim == 3:
            in_specs.append(pl.BlockSpec((2, b, H), lambda i: (0, i, 0)))
        else:
            in_specs.append(pl.BlockSpec((b, H), lambda i: (i, 0)))
    in_specs.append(pl.BlockSpec(w1.shape, lambda i: (0, 0)))
    out_shape = [jax.ShapeDtypeStruct((m, dout), jnp.float32),
                 jax.ShapeDtypeStruct((2, dout), jnp.float32)]
    out_specs = [pl.BlockSpec((b, dout), lambda i: (i, 0)),
                 pl.BlockSpec((2, dout), lambda i: (0, 0))]
    return pl.pallas_call(
        functools.partial(_k1_body, groups, len(parts)),
        grid=(m // b,),
        in_specs=in_specs,
        out_specs=out_specs,
        out_shape=out_shape,
    )(*parts, w1)


def _bn_coeffs(s_ref, g_ref, b_ref, inv_m):
    mean = s_ref[0:1, :] * inv_m
    var = s_ref[1:2, :] * inv_m - mean * mean
    scale = g_ref[...] * jax.lax.rsqrt(var + EPS)
    shift = b_ref[...] - mean * scale
    return scale, shift


def _k2_body(inv_m, u_ref, s_ref, g_ref, b_ref, w_ref, v_ref, sv_ref):
    scale, shift = _bn_coeffs(s_ref, g_ref, b_ref, inv_m)
    h = jnp.maximum(u_ref[...] * scale + shift, 0.0)
    v = jax.lax.dot_general(_bf(h), _bf(w_ref[...]), (((1,), (0,)), ((), ())),
                            preferred_element_type=jnp.float32,
                                precision=jax.lax.Precision.HIGHEST)
    v_ref[...] = v
    s1 = jnp.sum(v, axis=0, keepdims=True)
    s2 = jnp.sum(v * v, axis=0, keepdims=True)
    st = jnp.concatenate([s1, s2], axis=0)

    @pl.when(pl.program_id(0) == 0)
    def _():
        sv_ref[...] = st

    @pl.when(pl.program_id(0) != 0)
    def _():
        sv_ref[...] = sv_ref[...] + st


def bn_matmul_stats(u, s_u, g, bb, w2):
    m, din = u.shape
    dout = w2.shape[1]
    b = _pick_block(m)
    return pl.pallas_call(
        functools.partial(_k2_body, 1.0 / m),
        grid=(m // b,),
        in_specs=[pl.BlockSpec((b, din), lambda i: (i, 0)),
                  pl.BlockSpec((2, din), lambda i: (0, 0)),
                  pl.BlockSpec((1, din), lambda i: (0, 0)),
                  pl.BlockSpec((1, din), lambda i: (0, 0)),
                  pl.BlockSpec((din, dout), lambda i: (0, 0))],
        out_specs=[pl.BlockSpec((b, dout), lambda i: (i, 0)),
                   pl.BlockSpec((2, dout), lambda i: (0, 0))],
        out_shape=[jax.ShapeDtypeStruct((m, dout), jnp.float32),
                   jax.ShapeDtypeStruct((2, dout), jnp.float32)],
    )(u, s_u, g.reshape(1, din), bb.reshape(1, din), w2)


def _k3_body(inv_m, v_ref, s_ref, g_ref, b_ref, y_ref):
    scale, shift = _bn_coeffs(s_ref, g_ref, b_ref, inv_m)
    y_ref[...] = jnp.maximum(v_ref[...] * scale + shift, 0.0)


def bn_act(v, s_v, g, bb):
    m, d = v.shape
    b = _pick_block(m)
    return pl.pallas_call(
        functools.partial(_k3_body, 1.0 / m),
        grid=(m // b,),
        in_specs=[pl.BlockSpec((b, d), lambda i: (i, 0)),
                  pl.BlockSpec((2, d), lambda i: (0, 0)),
                  pl.BlockSpec((1, d), lambda i: (0, 0)),
                  pl.BlockSpec((1, d), lambda i: (0, 0))],
        out_specs=pl.BlockSpec((b, d), lambda i: (i, 0)),
        out_shape=jax.ShapeDtypeStruct((m, d), jnp.float32),
    )(v, s_v, g.reshape(1, d), bb.reshape(1, d))


def _k1f_body(inv_m, v0_ref, s0_ref, g0_ref, b0_ref,
              v2_ref, s2_ref, g2_ref, b2_ref, w_ref, u_ref, s_ref):
    sc0, sh0 = _bn_coeffs(s0_ref, g0_ref, b0_ref, inv_m)
    sc2, sh2 = _bn_coeffs(s2_ref, g2_ref, b2_ref, inv_m)
    x0 = jnp.maximum(v0_ref[...] * sc0 + sh0, 0.0)
    x2 = jnp.maximum(v2_ref[...] * sc2 + sh2, 0.0)
    u = (jax.lax.dot_general(_bf(x0), _bf(w_ref[pl.ds(0, H), :]),
                             (((1,), (0,)), ((), ())),
                             preferred_element_type=jnp.float32,
                                precision=jax.lax.Precision.HIGHEST)
         + jax.lax.dot_general(_bf(x2), _bf(w_ref[pl.ds(H, H), :]),
                               (((1,), (0,)), ((), ())),
                               preferred_element_type=jnp.float32,
                                precision=jax.lax.Precision.HIGHEST))
    u_ref[...] = u
    s1 = jnp.sum(u, axis=0, keepdims=True)
    s2 = jnp.sum(u * u, axis=0, keepdims=True)
    st = jnp.concatenate([s1, s2], axis=0)

    @pl.when(pl.program_id(0) == 0)
    def _():
        s_ref[...] = st

    @pl.when(pl.program_id(0) != 0)
    def _():
        s_ref[...] = s_ref[...] + st


def bn2_matmul_stats(v0, s0, g0, b0, v2, s2, g2, b2, w1):
    m = v0.shape[0]
    b = _pick_block(m)
    dout = w1.shape[1]
    sm = pl.BlockSpec((2, H), lambda i: (0, 0))
    gm = pl.BlockSpec((1, H), lambda i: (0, 0))
    return pl.pallas_call(
        functools.partial(_k1f_body, 1.0 / m),
        grid=(m // b,),
        in_specs=[pl.BlockSpec((b, H), lambda i: (i, 0)), sm, gm, gm,
                  pl.BlockSpec((b, H), lambda i: (i, 0)), sm, gm, gm,
                  pl.BlockSpec((2 * H, dout), lambda i: (0, 0))],
        out_specs=[pl.BlockSpec((b, dout), lambda i: (i, 0)),
                   pl.BlockSpec((2, dout), lambda i: (0, 0))],
        out_shape=[jax.ShapeDtypeStruct((m, dout), jnp.float32),
                   jax.ShapeDtypeStruct((2, dout), jnp.float32)],
    )(v0, s0, g0.reshape(1, H), b0.reshape(1, H),
      v2, s2, g2.reshape(1, H), b2.reshape(1, H), w1)


def _add2_body(x_ref, o_ref):
    o_ref[...] = x_ref[0] + x_ref[1]


def add_halves(x2):
    _, t, d = x2.shape
    b = _pick_block(t)
    return pl.pallas_call(
        _add2_body,
        grid=(t // b,),
        in_specs=[pl.BlockSpec((2, b, d), lambda i: (0, i, 0))],
        out_specs=pl.BlockSpec((b, d), lambda i: (i, 0)),
        out_shape=jax.ShapeDtypeStruct((t, d), jnp.float32),
    )(x2)


def sc_gather_sum(table, idx0, idx1):
    return table[idx0] + table[idx1]


def sc_gather(table, idx):
    return table[idx]


def sc_scatter_add(rows, idxs, t):
    acc = jnp.zeros((t, H), jnp.float32)
    for ix in idxs:
        acc = acc.at[ix].add(rows)
    return jnp.stack([acc, jnp.zeros((t, H), jnp.float32)])


def _mlp(parts, groups, p):
    u, su = matmul_stats(parts, p["W1"], groups)
    v, sv = bn_matmul_stats(u, su, p["g1"], p["b1"], p["W2"])
    return bn_act(v, sv, p["g2"], p["b2"])


def kernel(node_rep, edge_rep, cycle_rep, edge_index, cycle_node_ids,
           cycle_ids, node_mlp, edge_mlp0, cycle_mlp, edge_mlpc, edge_mlpt):
    n = node_rep.shape[0]
    c = 16000
    src, dst = edge_index[0], edge_index[1]

    n2e = sc_gather_sum(node_rep, src, dst)
    u0, su0 = matmul_stats([edge_rep, n2e], edge_mlp0["W1"], [[0], [1]])
    v0, sv0 = bn_matmul_stats(u0, su0, edge_mlp0["g1"], edge_mlp0["b1"],
                              edge_mlp0["W2"])
    edge_out0 = bn_act(v0, sv0, edge_mlp0["g2"], edge_mlp0["b2"])

    e2n = sc_scatter_add(edge_out0, [src, dst], n)
    node_out = _mlp([node_rep, e2n], [[0], [1]], node_mlp)

    e_at_n = add_halves(sc_scatter_add(edge_rep, [src, dst], n))
    per_row = sc_gather(e_at_n, cycle_node_ids)
    cyc_sum = add_halves(sc_scatter_add(per_row, [cycle_ids], c))
    gcyc = sc_gather(cyc_sum, cycle_ids)
    cycle_out = _mlp([cycle_rep, per_row, gcyc], [[0], [1], [2]], cycle_mlp)

    c_at_n = add_halves(sc_scatter_add(cycle_out, [cycle_node_ids], n))
    c2e = sc_gather_sum(c_at_n, src, dst)
    uc, suc = matmul_stats([edge_rep, c2e], edge_mlpc["W1"], [[0], [1]])
    vc, svc = bn_matmul_stats(uc, suc, edge_mlpc["g1"], edge_mlpc["b1"],
                              edge_mlpc["W2"])

    ut, sut = bn2_matmul_stats(v0, sv0, edge_mlp0["g2"], edge_mlp0["b2"],
                               vc, svc, edge_mlpc["g2"], edge_mlpc["b2"],
                               edge_mlpt["W1"])
    vt, svt = bn_matmul_stats(ut, sut, edge_mlpt["g1"], edge_mlpt["b1"],
                              edge_mlpt["W2"])
    edge_out = bn_act(vt, svt, edge_mlpt["g2"], edge_mlpt["b2"])

    return (node_out, edge_out, cycle_out)


# speedup vs baseline: 2.0285x; 2.0285x over previous
import functools

import jax
import jax.numpy as jnp
from jax import lax
from jax.experimental import pallas as pl
from jax.experimental.pallas import tpu as pltpu

H = 128
EPS = 1e-5


def _pick_block(m):
    for b in (2000, 1000, 500, 250, 125, 100, 40, 8):
        if m % b == 0:
            return b
    return m


def _bf(x):
    return x


def _k1_body(groups, nparts, *refs):
    part_refs = refs[:nparts]
    w_ref = refs[nparts]
    u_ref, s_ref = refs[nparts + 1], refs[nparts + 2]
    acc = None
    for gi, members in enumerate(groups):
        xg = None
        for pix in members:
            r = part_refs[pix]
            if len(r.shape) == 3:
                xv = r[0] + r[1]
            else:
                xv = r[...]
            xg = xv if xg is None else xg + xv
        wg = w_ref[pl.ds(gi * H, H), :]
        p = jax.lax.dot_general(_bf(xg), _bf(wg), (((1,), (0,)), ((), ())),
                                preferred_element_type=jnp.float32,
                                precision=jax.lax.Precision.HIGHEST)
        acc = p if acc is None else acc + p
    u_ref[...] = acc
    s1 = jnp.sum(acc, axis=0, keepdims=True)
    s2 = jnp.sum(acc * acc, axis=0, keepdims=True)
    st = jnp.concatenate([s1, s2], axis=0)

    @pl.when(pl.program_id(0) == 0)
    def _():
        s_ref[...] = st

    @pl.when(pl.program_id(0) != 0)
    def _():
        s_ref[...] = s_ref[...] + st


def matmul_stats(parts, w1, groups):
    m = parts[0].shape[-2]
    b = _pick_block(m)
    dout = w1.shape[1]
    in_specs = []
    for p in parts:
        if p.ndim == 3:
            in_specs.append(pl.BlockSpec((2, b, H), lambda i: (0, i, 0)))
        else:
            in_specs.append(pl.BlockSpec((b, H), lambda i: (i, 0)))
    in_specs.append(pl.BlockSpec(w1.shape, lambda i: (0, 0)))
    out_shape = [jax.ShapeDtypeStruct((m, dout), jnp.float32),
                 jax.ShapeDtypeStruct((2, dout), jnp.float32)]
    out_specs = [pl.BlockSpec((b, dout), lambda i: (i, 0)),
                 pl.BlockSpec((2, dout), lambda i: (0, 0))]
    return pl.pallas_call(
        functools.partial(_k1_body, groups, len(parts)),
        grid=(m // b,),
        in_specs=in_specs,
        out_specs=out_specs,
        out_shape=out_shape,
    )(*parts, w1)


def _bn_coeffs(s_ref, g_ref, b_ref, inv_m):
    mean = s_ref[0:1, :] * inv_m
    var = s_ref[1:2, :] * inv_m - mean * mean
    scale = g_ref[...] * jax.lax.rsqrt(var + EPS)
    shift = b_ref[...] - mean * scale
    return scale, shift


def _k2_body(inv_m, u_ref, s_ref, g_ref, b_ref, w_ref, v_ref, sv_ref):
    scale, shift = _bn_coeffs(s_ref, g_ref, b_ref, inv_m)
    h = jnp.maximum(u_ref[...] * scale + shift, 0.0)
    v = jax.lax.dot_general(_bf(h), _bf(w_ref[...]), (((1,), (0,)), ((), ())),
                            preferred_element_type=jnp.float32,
                                precision=jax.lax.Precision.HIGHEST)
    v_ref[...] = v
    s1 = jnp.sum(v, axis=0, keepdims=True)
    s2 = jnp.sum(v * v, axis=0, keepdims=True)
    st = jnp.concatenate([s1, s2], axis=0)

    @pl.when(pl.program_id(0) == 0)
    def _():
        sv_ref[...] = st

    @pl.when(pl.program_id(0) != 0)
    def _():
        sv_ref[...] = sv_ref[...] + st


def bn_matmul_stats(u, s_u, g, bb, w2):
    m, din = u.shape
    dout = w2.shape[1]
    b = _pick_block(m)
    return pl.pallas_call(
        functools.partial(_k2_body, 1.0 / m),
        grid=(m // b,),
        in_specs=[pl.BlockSpec((b, din), lambda i: (i, 0)),
                  pl.BlockSpec((2, din), lambda i: (0, 0)),
                  pl.BlockSpec((1, din), lambda i: (0, 0)),
                  pl.BlockSpec((1, din), lambda i: (0, 0)),
                  pl.BlockSpec((din, dout), lambda i: (0, 0))],
        out_specs=[pl.BlockSpec((b, dout), lambda i: (i, 0)),
                   pl.BlockSpec((2, dout), lambda i: (0, 0))],
        out_shape=[jax.ShapeDtypeStruct((m, dout), jnp.float32),
                   jax.ShapeDtypeStruct((2, dout), jnp.float32)],
    )(u, s_u, g.reshape(1, din), bb.reshape(1, din), w2)


def _k3_body(inv_m, v_ref, s_ref, g_ref, b_ref, y_ref):
    scale, shift = _bn_coeffs(s_ref, g_ref, b_ref, inv_m)
    y_ref[...] = jnp.maximum(v_ref[...] * scale + shift, 0.0)


def bn_act(v, s_v, g, bb):
    m, d = v.shape
    b = _pick_block(m)
    return pl.pallas_call(
        functools.partial(_k3_body, 1.0 / m),
        grid=(m // b,),
        in_specs=[pl.BlockSpec((b, d), lambda i: (i, 0)),
                  pl.BlockSpec((2, d), lambda i: (0, 0)),
                  pl.BlockSpec((1, d), lambda i: (0, 0)),
                  pl.BlockSpec((1, d), lambda i: (0, 0))],
        out_specs=pl.BlockSpec((b, d), lambda i: (i, 0)),
        out_shape=jax.ShapeDtypeStruct((m, d), jnp.float32),
    )(v, s_v, g.reshape(1, d), bb.reshape(1, d))


def _k1f_body(inv_m, v0_ref, s0_ref, g0_ref, b0_ref,
              v2_ref, s2_ref, g2_ref, b2_ref, w_ref, u_ref, s_ref):
    sc0, sh0 = _bn_coeffs(s0_ref, g0_ref, b0_ref, inv_m)
    sc2, sh2 = _bn_coeffs(s2_ref, g2_ref, b2_ref, inv_m)
    x0 = jnp.maximum(v0_ref[...] * sc0 + sh0, 0.0)
    x2 = jnp.maximum(v2_ref[...] * sc2 + sh2, 0.0)
    u = (jax.lax.dot_general(_bf(x0), _bf(w_ref[pl.ds(0, H), :]),
                             (((1,), (0,)), ((), ())),
                             preferred_element_type=jnp.float32,
                                precision=jax.lax.Precision.HIGHEST)
         + jax.lax.dot_general(_bf(x2), _bf(w_ref[pl.ds(H, H), :]),
                               (((1,), (0,)), ((), ())),
                               preferred_element_type=jnp.float32,
                                precision=jax.lax.Precision.HIGHEST))
    u_ref[...] = u
    s1 = jnp.sum(u, axis=0, keepdims=True)
    s2 = jnp.sum(u * u, axis=0, keepdims=True)
    st = jnp.concatenate([s1, s2], axis=0)

    @pl.when(pl.program_id(0) == 0)
    def _():
        s_ref[...] = st

    @pl.when(pl.program_id(0) != 0)
    def _():
        s_ref[...] = s_ref[...] + st


def bn2_matmul_stats(v0, s0, g0, b0, v2, s2, g2, b2, w1):
    m = v0.shape[0]
    b = _pick_block(m)
    dout = w1.shape[1]
    sm = pl.BlockSpec((2, H), lambda i: (0, 0))
    gm = pl.BlockSpec((1, H), lambda i: (0, 0))
    return pl.pallas_call(
        functools.partial(_k1f_body, 1.0 / m),
        grid=(m // b,),
        in_specs=[pl.BlockSpec((b, H), lambda i: (i, 0)), sm, gm, gm,
                  pl.BlockSpec((b, H), lambda i: (i, 0)), sm, gm, gm,
                  pl.BlockSpec((2 * H, dout), lambda i: (0, 0))],
        out_specs=[pl.BlockSpec((b, dout), lambda i: (i, 0)),
                   pl.BlockSpec((2, dout), lambda i: (0, 0))],
        out_shape=[jax.ShapeDtypeStruct((m, dout), jnp.float32),
                   jax.ShapeDtypeStruct((2, dout), jnp.float32)],
    )(v0, s0, g0.reshape(1, H), b0.reshape(1, H),
      v2, s2, g2.reshape(1, H), b2.reshape(1, H), w1)


def _add2_body(x_ref, o_ref):
    o_ref[...] = x_ref[0] + x_ref[1]


def add_halves(x2):
    _, t, d = x2.shape
    b = _pick_block(t)
    return pl.pallas_call(
        _add2_body,
        grid=(t // b,),
        in_specs=[pl.BlockSpec((2, b, d), lambda i: (0, i, 0))],
        out_specs=pl.BlockSpec((b, d), lambda i: (i, 0)),
        out_shape=jax.ShapeDtypeStruct((t, d), jnp.float32),
    )(x2)


from jax.experimental.pallas import tpu_sc as plsc

_SC_MESH = plsc.VectorSubcoreMesh(core_axis_name="core",
                                  subcore_axis_name="subcore")


def _win(m, cap=128):
    for w in (128, 88, 80, 64, 40, 16):
        if w <= cap and m % w == 0:
            return w
    raise ValueError(m)


def sc_gather2(table, idx2):
    m = idx2.shape[1]
    w = _win(m)
    idx3 = jnp.stack([idx2[0].reshape(m // w, w), idx2[1].reshape(m // w, w)],
                     axis=1)

    @functools.partial(
        pl.kernel,
        out_type=jax.ShapeDtypeStruct((2, m, H), jnp.float32),
        mesh=_SC_MESH,
        scratch_types=[pltpu.SemaphoreType.DMA, pltpu.SemaphoreType.DMA],
    )
    def k(table_hbm, idx_hbm, out_hbm, sem0, sem1):
        def body(idx_vmem, out_vmem):
            c0 = pltpu.async_copy(table_hbm.at[idx_vmem.at[0, 0]],
                                  out_vmem.at[0], sem0)
            c1 = pltpu.async_copy(table_hbm.at[idx_vmem.at[0, 1]],
                                  out_vmem.at[1], sem1)
            c0.wait()
            c1.wait()

        pltpu.emit_pipeline(
            body,
            grid=(m // w,),
            in_specs=[pl.BlockSpec((1, 2, w), lambda i: (i, 0, 0))],
            out_specs=[pl.BlockSpec((2, w, H), lambda i: (0, i, 0))],
            core_axis_name=("core", "subcore"),
            dimension_semantics=(pltpu.PARALLEL,),
        )(idx_hbm, out_hbm)

    return k(table, idx3)


def sc_gather(table, idx):
    m = idx.shape[0]
    w = _win(m)
    idx3 = idx.reshape(m // w, 1, w)

    @functools.partial(
        pl.kernel,
        out_type=jax.ShapeDtypeStruct((m, H), jnp.float32),
        mesh=_SC_MESH,
        scratch_types=[pltpu.SemaphoreType.DMA],
    )
    def k(table_hbm, idx_hbm, out_hbm, sem):
        def body(idx_vmem, out_vmem):
            pltpu.async_copy(table_hbm.at[idx_vmem.at[0, 0]], out_vmem,
                             sem).wait()

        pltpu.emit_pipeline(
            body,
            grid=(m // w,),
            in_specs=[pl.BlockSpec((1, 1, w), lambda i: (i, 0, 0))],
            out_specs=[pl.BlockSpec((w, H), lambda i: (i, 0))],
            core_axis_name=("core", "subcore"),
            dimension_semantics=(pltpu.PARALLEL,),
        )(idx_hbm, out_hbm)

    return k(table, idx3)


def sc_scatter_add(rows, idx, t):
    kn, m = idx.shape
    w = _win(m)
    idx3 = jnp.stack([idx[j].reshape(m // w, w) for j in range(kn)], axis=1)
    cz = 80
    nch = t // cz
    per_tile = -(-nch // 16)
    zrs = jnp.zeros((cz, H), jnp.float32)

    @functools.partial(
        pl.kernel,
        out_type=jax.ShapeDtypeStruct((2, t, H), jnp.float32),
        mesh=_SC_MESH,
        scratch_types=[pltpu.VMEM_SHARED((t, H), jnp.float32)],
    )
    def k(rows_hbm, idx_hbm, z_hbm, out_hbm, acc):
        cid = lax.axis_index("core")
        sid = lax.axis_index("subcore")

        @pl.loop(0, per_tile)
        def _(i):
            ch = i * 16 + sid

            @pl.when(ch < nch)
            def _():
                off = pl.multiple_of(ch * cz, 8)
                pltpu.sync_copy(z_hbm, acc.at[pl.ds(off, cz)])

        plsc.subcore_barrier()

        def body(rows_vmem, idx_vmem):
            for j in range(kn):
                pltpu.sync_copy(rows_vmem, acc.at[idx_vmem.at[0, j]],
                                add=True)

        pltpu.emit_pipeline(
            body,
            grid=(m // w,),
            in_specs=[pl.BlockSpec((w, H), lambda i: (i, 0)),
                      pl.BlockSpec((1, kn, w), lambda i: (i, 0, 0))],
            out_specs=[],
            core_axis_name=("core", "subcore"),
            dimension_semantics=(pltpu.PARALLEL,),
        )(rows_hbm, idx_hbm)

        plsc.subcore_barrier()

        @pl.loop(0, per_tile)
        def _(i):
            ch = i * 16 + sid

            @pl.when(ch < nch)
            def _():
                off = pl.multiple_of(ch * cz, 8)
                pltpu.sync_copy(acc.at[pl.ds(off, cz)],
                                out_hbm.at[cid].at[pl.ds(off, cz)])

    return k(rows, idx3, zrs)


def sc_segment_sum_c(rows, idx, c):
    m = idx.shape[0]
    w = 80
    assert m % w == 0 and c % 2 == 0
    half = c // 2
    pad = half + 80
    idx3 = idx.reshape(m // w, 1, w)
    cz = 80
    nzch = pad // cz
    noch = half // cz
    zrs = jnp.zeros((cz, H), jnp.float32)

    @functools.partial(
        pl.kernel,
        out_type=jax.ShapeDtypeStruct((2, half, H), jnp.float32),
        mesh=_SC_MESH,
        scratch_types=[pltpu.VMEM_SHARED((pad, H), jnp.float32),
                       pltpu.VMEM((1, w), jnp.int32)],
    )
    def k(rows_hbm, idx_hbm, z_hbm, out_hbm, acc, sidx):
        cid = lax.axis_index("core")
        sid = lax.axis_index("subcore")
        lo = cid * half

        @pl.loop(0, -(-nzch // 16))
        def _(i):
            ch = i * 16 + sid

            @pl.when(ch < nzch)
            def _():
                off = pl.multiple_of(ch * cz, 8)
                pltpu.sync_copy(z_hbm, acc.at[pl.ds(off, cz)])

        plsc.subcore_barrier()

        def body(rows_vmem, idx_vmem):
            for kk in range(w // 16):
                v = idx_vmem[0, 0, pl.ds(kk * 16, 16)]
                ok = (v >= lo) & (v < lo + half)
                sidx[0, pl.ds(kk * 16, 16)] = jnp.where(ok, v - lo, half)
            pltpu.sync_copy(rows_vmem, acc.at[sidx.at[0]], add=True)

        pltpu.emit_pipeline(
            body,
            grid=(m // w,),
            in_specs=[pl.BlockSpec((w, H), lambda i: (i, 0)),
                      pl.BlockSpec((1, 1, w), lambda i: (i, 0, 0))],
            out_specs=[],
            core_axis_name="subcore",
            dimension_semantics=(pltpu.PARALLEL,),
        )(rows_hbm, idx_hbm)

        plsc.subcore_barrier()

        @pl.loop(0, -(-noch // 16))
        def _(i):
            ch = i * 16 + sid

            @pl.when(ch < noch)
            def _():
                off = pl.multiple_of(ch * cz, 8)
                pltpu.sync_copy(acc.at[pl.ds(off, cz)],
                                out_hbm.at[cid].at[pl.ds(off, cz)])

    return k(rows, idx3, zrs).reshape(c, H)


def _mlp(parts, groups, p):
    u, su = matmul_stats(parts, p["W1"], groups)
    v, sv = bn_matmul_stats(u, su, p["g1"], p["b1"], p["W2"])
    return bn_act(v, sv, p["g2"], p["b2"])


def kernel(node_rep, edge_rep, cycle_rep, edge_index, cycle_node_ids,
           cycle_ids, node_mlp, edge_mlp0, cycle_mlp, edge_mlpc, edge_mlpt):
    n = node_rep.shape[0]
    c = 16000
    src, dst = edge_index[0], edge_index[1]

    n2e = sc_gather2(node_rep, edge_index)
    u0, su0 = matmul_stats([edge_rep, n2e], edge_mlp0["W1"], [[0], [1]])
    v0, sv0 = bn_matmul_stats(u0, su0, edge_mlp0["g1"], edge_mlp0["b1"],
                              edge_mlp0["W2"])
    edge_out0 = bn_act(v0, sv0, edge_mlp0["g2"], edge_mlp0["b2"])

    e2n = sc_scatter_add(edge_out0, edge_index, n)
    node_out = _mlp([node_rep, e2n], [[0], [1]], node_mlp)

    e_at_n = add_halves(sc_scatter_add(edge_rep, edge_index, n))
    per_row = sc_gather(e_at_n, cycle_node_ids)
    cyc_sum = sc_segment_sum_c(per_row, cycle_ids, c)
    gcyc = sc_gather(cyc_sum, cycle_ids)
    cycle_out = _mlp([cycle_rep, per_row, gcyc], [[0], [1], [2]], cycle_mlp)

    c_at_n = add_halves(sc_scatter_add(cycle_out,
                                       cycle_node_ids.reshape(1, -1), n))
    c2e = sc_gather2(c_at_n, edge_index)
    uc, suc = matmul_stats([edge_rep, c2e], edge_mlpc["W1"], [[0], [1]])
    vc, svc = bn_matmul_stats(uc, suc, edge_mlpc["g1"], edge_mlpc["b1"],
                              edge_mlpc["W2"])

    ut, sut = bn2_matmul_stats(v0, sv0, edge_mlp0["g2"], edge_mlp0["b2"],
                               vc, svc, edge_mlpc["g2"], edge_mlpc["b2"],
                               edge_mlpt["W1"])
    vt, svt = bn_matmul_stats(ut, sut, edge_mlpt["g1"], edge_mlpt["b1"],
                              edge_mlpt["W2"])
    edge_out = bn_act(vt, svt, edge_mlpt["g2"], edge_mlpt["b2"])

    return (node_out, edge_out, cycle_out)


# speedup vs baseline: 2.7984x; 1.3795x over previous
import functools

import jax
import jax.numpy as jnp
from jax import lax
from jax.experimental import pallas as pl
from jax.experimental.pallas import tpu as pltpu

H = 128
EPS = 1e-5


def _pick_block(m):
    for b in (2000, 1000, 500, 250, 125, 100, 40, 8):
        if m % b == 0:
            return b
    return m


def _bf(x):
    return x


def _k1_body(groups, nparts, *refs):
    part_refs = refs[:nparts]
    w_ref = refs[nparts]
    u_ref, s_ref = refs[nparts + 1], refs[nparts + 2]
    acc = None
    for gi, members in enumerate(groups):
        xg = None
        for pix in members:
            r = part_refs[pix]
            if len(r.shape) == 3:
                xv = r[0] + r[1]
            else:
                xv = r[...]
            xg = xv if xg is None else xg + xv
        wg = w_ref[pl.ds(gi * H, H), :]
        p = jax.lax.dot_general(_bf(xg), _bf(wg), (((1,), (0,)), ((), ())),
                                preferred_element_type=jnp.float32)
        acc = p if acc is None else acc + p
    u_ref[...] = acc
    s1 = jnp.sum(acc, axis=0, keepdims=True)
    s2 = jnp.sum(acc * acc, axis=0, keepdims=True)
    st = jnp.concatenate([s1, s2], axis=0)

    @pl.when(pl.program_id(0) == 0)
    def _():
        s_ref[...] = st

    @pl.when(pl.program_id(0) != 0)
    def _():
        s_ref[...] = s_ref[...] + st


def matmul_stats(parts, w1, groups):
    m = parts[0].shape[-2]
    b = _pick_block(m)
    dout = w1.shape[1]
    in_specs = []
    for p in parts:
        if p.ndim == 3:
            in_specs.append(pl.BlockSpec((2, b, H), lambda i: (0, i, 0)))
        else:
            in_specs.append(pl.BlockSpec((b, H), lambda i: (i, 0)))
    in_specs.append(pl.BlockSpec(w1.shape, lambda i: (0, 0)))
    out_shape = [jax.ShapeDtypeStruct((m, dout), jnp.float32),
                 jax.ShapeDtypeStruct((2, dout), jnp.float32)]
    out_specs = [pl.BlockSpec((b, dout), lambda i: (i, 0)),
                 pl.BlockSpec((2, dout), lambda i: (0, 0))]
    return pl.pallas_call(
        functools.partial(_k1_body, groups, len(parts)),
        grid=(m // b,),
        in_specs=in_specs,
        out_specs=out_specs,
        out_shape=out_shape,
    )(*parts, w1)


def _bn_coeffs(s_ref, g_ref, b_ref, inv_m):
    mean = s_ref[0:1, :] * inv_m
    var = s_ref[1:2, :] * inv_m - mean * mean
    scale = g_ref[...] * jax.lax.rsqrt(var + EPS)
    shift = b_ref[...] - mean * scale
    return scale, shift


def _k2_body(inv_m, u_ref, s_ref, g_ref, b_ref, w_ref, v_ref, sv_ref):
    scale, shift = _bn_coeffs(s_ref, g_ref, b_ref, inv_m)
    h = jnp.maximum(u_ref[...] * scale + shift, 0.0)
    v = jax.lax.dot_general(_bf(h), _bf(w_ref[...]), (((1,), (0,)), ((), ())),
                            preferred_element_type=jnp.float32)
    v_ref[...] = v
    s1 = jnp.sum(v, axis=0, keepdims=True)
    s2 = jnp.sum(v * v, axis=0, keepdims=True)
    st = jnp.concatenate([s1, s2], axis=0)

    @pl.when(pl.program_id(0) == 0)
    def _():
        sv_ref[...] = st

    @pl.when(pl.program_id(0) != 0)
    def _():
        sv_ref[...] = sv_ref[...] + st


def bn_matmul_stats(u, s_u, g, bb, w2):
    m, din = u.shape
    dout = w2.shape[1]
    b = _pick_block(m)
    return pl.pallas_call(
        functools.partial(_k2_body, 1.0 / m),
        grid=(m // b,),
        in_specs=[pl.BlockSpec((b, din), lambda i: (i, 0)),
                  pl.BlockSpec((2, din), lambda i: (0, 0)),
                  pl.BlockSpec((1, din), lambda i: (0, 0)),
                  pl.BlockSpec((1, din), lambda i: (0, 0)),
                  pl.BlockSpec((din, dout), lambda i: (0, 0))],
        out_specs=[pl.BlockSpec((b, dout), lambda i: (i, 0)),
                   pl.BlockSpec((2, dout), lambda i: (0, 0))],
        out_shape=[jax.ShapeDtypeStruct((m, dout), jnp.float32),
                   jax.ShapeDtypeStruct((2, dout), jnp.float32)],
    )(u, s_u, g.reshape(1, din), bb.reshape(1, din), w2)


def _k3_body(inv_m, v_ref, s_ref, g_ref, b_ref, y_ref):
    scale, shift = _bn_coeffs(s_ref, g_ref, b_ref, inv_m)
    y_ref[...] = jnp.maximum(v_ref[...] * scale + shift, 0.0)


def bn_act(v, s_v, g, bb):
    m, d = v.shape
    b = _pick_block(m)
    return pl.pallas_call(
        functools.partial(_k3_body, 1.0 / m),
        grid=(m // b,),
        in_specs=[pl.BlockSpec((b, d), lambda i: (i, 0)),
                  pl.BlockSpec((2, d), lambda i: (0, 0)),
                  pl.BlockSpec((1, d), lambda i: (0, 0)),
                  pl.BlockSpec((1, d), lambda i: (0, 0))],
        out_specs=pl.BlockSpec((b, d), lambda i: (i, 0)),
        out_shape=jax.ShapeDtypeStruct((m, d), jnp.float32),
    )(v, s_v, g.reshape(1, d), bb.reshape(1, d))


def _k1f_body(inv_m, v0_ref, s0_ref, g0_ref, b0_ref,
              v2_ref, s2_ref, g2_ref, b2_ref, w_ref, u_ref, s_ref):
    sc0, sh0 = _bn_coeffs(s0_ref, g0_ref, b0_ref, inv_m)
    sc2, sh2 = _bn_coeffs(s2_ref, g2_ref, b2_ref, inv_m)
    x0 = jnp.maximum(v0_ref[...] * sc0 + sh0, 0.0)
    x2 = jnp.maximum(v2_ref[...] * sc2 + sh2, 0.0)
    u = (jax.lax.dot_general(_bf(x0), _bf(w_ref[pl.ds(0, H), :]),
                             (((1,), (0,)), ((), ())),
                             preferred_element_type=jnp.float32)
         + jax.lax.dot_general(_bf(x2), _bf(w_ref[pl.ds(H, H), :]),
                               (((1,), (0,)), ((), ())),
                               preferred_element_type=jnp.float32))
    u_ref[...] = u
    s1 = jnp.sum(u, axis=0, keepdims=True)
    s2 = jnp.sum(u * u, axis=0, keepdims=True)
    st = jnp.concatenate([s1, s2], axis=0)

    @pl.when(pl.program_id(0) == 0)
    def _():
        s_ref[...] = st

    @pl.when(pl.program_id(0) != 0)
    def _():
        s_ref[...] = s_ref[...] + st


def bn2_matmul_stats(v0, s0, g0, b0, v2, s2, g2, b2, w1):
    m = v0.shape[0]
    b = _pick_block(m)
    dout = w1.shape[1]
    sm = pl.BlockSpec((2, H), lambda i: (0, 0))
    gm = pl.BlockSpec((1, H), lambda i: (0, 0))
    return pl.pallas_call(
        functools.partial(_k1f_body, 1.0 / m),
        grid=(m // b,),
        in_specs=[pl.BlockSpec((b, H), lambda i: (i, 0)), sm, gm, gm,
                  pl.BlockSpec((b, H), lambda i: (i, 0)), sm, gm, gm,
                  pl.BlockSpec((2 * H, dout), lambda i: (0, 0))],
        out_specs=[pl.BlockSpec((b, dout), lambda i: (i, 0)),
                   pl.BlockSpec((2, dout), lambda i: (0, 0))],
        out_shape=[jax.ShapeDtypeStruct((m, dout), jnp.float32),
                   jax.ShapeDtypeStruct((2, dout), jnp.float32)],
    )(v0, s0, g0.reshape(1, H), b0.reshape(1, H),
      v2, s2, g2.reshape(1, H), b2.reshape(1, H), w1)


def _add2_body(x_ref, o_ref):
    o_ref[...] = x_ref[0] + x_ref[1]


def add_halves(x2):
    _, t, d = x2.shape
    b = _pick_block(t)
    return pl.pallas_call(
        _add2_body,
        grid=(t // b,),
        in_specs=[pl.BlockSpec((2, b, d), lambda i: (0, i, 0))],
        out_specs=pl.BlockSpec((b, d), lambda i: (i, 0)),
        out_shape=jax.ShapeDtypeStruct((t, d), jnp.float32),
    )(x2)


from jax.experimental.pallas import tpu_sc as plsc

_SC_MESH = plsc.VectorSubcoreMesh(core_axis_name="core",
                                  subcore_axis_name="subcore")


def _win(m, cap=128):
    for w in (128, 88, 80, 64, 40, 16):
        if w <= cap and m % w == 0:
            return w
    raise ValueError(m)


def sc_gather2(table, idx2):
    m = idx2.shape[1]
    w = _win(m)
    idx3 = jnp.stack([idx2[0].reshape(m // w, w), idx2[1].reshape(m // w, w)],
                     axis=1)

    @functools.partial(
        pl.kernel,
        out_type=jax.ShapeDtypeStruct((2, m, H), jnp.float32),
        mesh=_SC_MESH,
        scratch_types=[pltpu.SemaphoreType.DMA, pltpu.SemaphoreType.DMA],
    )
    def k(table_hbm, idx_hbm, out_hbm, sem0, sem1):
        def body(idx_vmem, out_vmem):
            c0 = pltpu.async_copy(table_hbm.at[idx_vmem.at[0, 0]],
                                  out_vmem.at[0], sem0)
            c1 = pltpu.async_copy(table_hbm.at[idx_vmem.at[0, 1]],
                                  out_vmem.at[1], sem1)
            c0.wait()
            c1.wait()

        pltpu.emit_pipeline(
            body,
            grid=(m // w,),
            in_specs=[pl.BlockSpec((1, 2, w), lambda i: (i, 0, 0))],
            out_specs=[pl.BlockSpec((2, w, H), lambda i: (0, i, 0))],
            core_axis_name=("core", "subcore"),
            dimension_semantics=(pltpu.PARALLEL,),
        )(idx_hbm, out_hbm)

    return k(table, idx3)


def sc_gather(table, idx):
    m = idx.shape[0]
    w = _win(m)
    idx3 = idx.reshape(m // w, 1, w)

    @functools.partial(
        pl.kernel,
        out_type=jax.ShapeDtypeStruct((m, H), jnp.float32),
        mesh=_SC_MESH,
        scratch_types=[pltpu.SemaphoreType.DMA],
    )
    def k(table_hbm, idx_hbm, out_hbm, sem):
        def body(idx_vmem, out_vmem):
            pltpu.async_copy(table_hbm.at[idx_vmem.at[0, 0]], out_vmem,
                             sem).wait()

        pltpu.emit_pipeline(
            body,
            grid=(m // w,),
            in_specs=[pl.BlockSpec((1, 1, w), lambda i: (i, 0, 0))],
            out_specs=[pl.BlockSpec((w, H), lambda i: (i, 0))],
            core_axis_name=("core", "subcore"),
            dimension_semantics=(pltpu.PARALLEL,),
        )(idx_hbm, out_hbm)

    return k(table, idx3)


def sc_scatter_add(rows, idx, t):
    kn, m = idx.shape
    w = _win(m)
    idx3 = jnp.stack([idx[j].reshape(m // w, w) for j in range(kn)], axis=1)
    cz = 80
    nch = t // cz
    per_tile = -(-nch // 16)
    zrs = jnp.zeros((cz, H), jnp.float32)

    @functools.partial(
        pl.kernel,
        out_type=jax.ShapeDtypeStruct((2, t, H), jnp.float32),
        mesh=_SC_MESH,
        scratch_types=[pltpu.VMEM_SHARED((t, H), jnp.float32)],
    )
    def k(rows_hbm, idx_hbm, z_hbm, out_hbm, acc):
        cid = lax.axis_index("core")
        sid = lax.axis_index("subcore")

        @pl.loop(0, per_tile)
        def _(i):
            ch = i * 16 + sid

            @pl.when(ch < nch)
            def _():
                off = pl.multiple_of(ch * cz, 8)
                pltpu.sync_copy(z_hbm, acc.at[pl.ds(off, cz)])

        plsc.subcore_barrier()

        def body(rows_vmem, idx_vmem):
            for j in range(kn):
                pltpu.sync_copy(rows_vmem, acc.at[idx_vmem.at[0, j]],
                                add=True)

        pltpu.emit_pipeline(
            body,
            grid=(m // w,),
            in_specs=[pl.BlockSpec((w, H), lambda i: (i, 0)),
                      pl.BlockSpec((1, kn, w), lambda i: (i, 0, 0))],
            out_specs=[],
            core_axis_name=("core", "subcore"),
            dimension_semantics=(pltpu.PARALLEL,),
        )(rows_hbm, idx_hbm)

        plsc.subcore_barrier()

        @pl.loop(0, per_tile)
        def _(i):
            ch = i * 16 + sid

            @pl.when(ch < nch)
            def _():
                off = pl.multiple_of(ch * cz, 8)
                pltpu.sync_copy(acc.at[pl.ds(off, cz)],
                                out_hbm.at[cid].at[pl.ds(off, cz)])

    return k(rows, idx3, zrs)


def sc_segment_sum_c(rows, idx, c):
    m = idx.shape[0]
    w = 80
    assert m % w == 0 and c % 2 == 0
    half = c // 2
    pad = half + 80
    idx3 = idx.reshape(m // w, 1, w)
    cz = 80
    nzch = pad // cz
    noch = half // cz
    zrs = jnp.zeros((cz, H), jnp.float32)

    @functools.partial(
        pl.kernel,
        out_type=jax.ShapeDtypeStruct((2, half, H), jnp.float32),
        mesh=_SC_MESH,
        scratch_types=[pltpu.VMEM_SHARED((pad, H), jnp.float32),
                       pltpu.VMEM((1, w), jnp.int32)],
    )
    def k(rows_hbm, idx_hbm, z_hbm, out_hbm, acc, sidx):
        cid = lax.axis_index("core")
        sid = lax.axis_index("subcore")
        lo = cid * half

        @pl.loop(0, -(-nzch // 16))
        def _(i):
            ch = i * 16 + sid

            @pl.when(ch < nzch)
            def _():
                off = pl.multiple_of(ch * cz, 8)
                pltpu.sync_copy(z_hbm, acc.at[pl.ds(off, cz)])

        plsc.subcore_barrier()

        def body(rows_vmem, idx_vmem):
            for kk in range(w // 16):
                v = idx_vmem[0, 0, pl.ds(kk * 16, 16)]
                ok = (v >= lo) & (v < lo + half)
                sidx[0, pl.ds(kk * 16, 16)] = jnp.where(ok, v - lo, half)
            pltpu.sync_copy(rows_vmem, acc.at[sidx.at[0]], add=True)

        pltpu.emit_pipeline(
            body,
            grid=(m // w,),
            in_specs=[pl.BlockSpec((w, H), lambda i: (i, 0)),
                      pl.BlockSpec((1, 1, w), lambda i: (i, 0, 0))],
            out_specs=[],
            core_axis_name="subcore",
            dimension_semantics=(pltpu.PARALLEL,),
        )(rows_hbm, idx_hbm)

        plsc.subcore_barrier()

        @pl.loop(0, -(-noch // 16))
        def _(i):
            ch = i * 16 + sid

            @pl.when(ch < noch)
            def _():
                off = pl.multiple_of(ch * cz, 8)
                pltpu.sync_copy(acc.at[pl.ds(off, cz)],
                                out_hbm.at[cid].at[pl.ds(off, cz)])

    return k(rows, idx3, zrs).reshape(c, H)


def _mlp(parts, groups, p):
    u, su = matmul_stats(parts, p["W1"], groups)
    v, sv = bn_matmul_stats(u, su, p["g1"], p["b1"], p["W2"])
    return bn_act(v, sv, p["g2"], p["b2"])


def kernel(node_rep, edge_rep, cycle_rep, edge_index, cycle_node_ids,
           cycle_ids, node_mlp, edge_mlp0, cycle_mlp, edge_mlpc, edge_mlpt):
    n = node_rep.shape[0]
    c = 16000
    src, dst = edge_index[0], edge_index[1]

    n2e = sc_gather2(node_rep, edge_index)
    u0, su0 = matmul_stats([edge_rep, n2e], edge_mlp0["W1"], [[0], [1]])
    v0, sv0 = bn_matmul_stats(u0, su0, edge_mlp0["g1"], edge_mlp0["b1"],
                              edge_mlp0["W2"])
    edge_out0 = bn_act(v0, sv0, edge_mlp0["g2"], edge_mlp0["b2"])

    e_at_n = add_halves(sc_scatter_add(edge_rep, edge_index, n))
    per_row = sc_gather(e_at_n, cycle_node_ids)
    cyc_sum = sc_segment_sum_c(per_row, cycle_ids, c)
    gcyc = sc_gather(cyc_sum, cycle_ids)
    cycle_out = _mlp([cycle_rep, per_row, gcyc], [[0], [1], [2]], cycle_mlp)

    c_at_n = add_halves(sc_scatter_add(cycle_out,
                                       cycle_node_ids.reshape(1, -1), n))
    c2e = sc_gather2(c_at_n, edge_index)

    e2n = sc_scatter_add(edge_out0, edge_index, n)
    node_out = _mlp([node_rep, e2n], [[0], [1]], node_mlp)

    uc, suc = matmul_stats([edge_rep, c2e], edge_mlpc["W1"], [[0], [1]])
    vc, svc = bn_matmul_stats(uc, suc, edge_mlpc["g1"], edge_mlpc["b1"],
                              edge_mlpc["W2"])

    ut, sut = bn2_matmul_stats(v0, sv0, edge_mlp0["g2"], edge_mlp0["b2"],
                               vc, svc, edge_mlpc["g2"], edge_mlpc["b2"],
                               edge_mlpt["W1"])
    vt, svt = bn_matmul_stats(ut, sut, edge_mlpt["g1"], edge_mlpt["b1"],
                              edge_mlpt["W2"])
    edge_out = bn_act(vt, svt, edge_mlpt["g2"], edge_mlpt["b2"])

    return (node_out, edge_out, cycle_out)


# speedup vs baseline: 3.1670x; 1.1317x over previous
import functools

import jax
import jax.numpy as jnp
from jax import lax
from jax.experimental import pallas as pl
from jax.experimental.pallas import tpu as pltpu

H = 128
EPS = 1e-5


def _pick_block(m):
    for b in (4000, 2000, 1000, 500, 250, 125, 100, 40, 8):
        if m % b == 0:
            return b
    return m


def _bf(x):
    return x


def _k1_body(groups, nparts, *refs):
    part_refs = refs[:nparts]
    w_ref = refs[nparts]
    u_ref, s_ref = refs[nparts + 1], refs[nparts + 2]
    acc = None
    for gi, members in enumerate(groups):
        xg = None
        for pix in members:
            r = part_refs[pix]
            if len(r.shape) == 3:
                xv = r[0] + r[1]
            else:
                xv = r[...]
            xg = xv if xg is None else xg + xv
        wg = w_ref[pl.ds(gi * H, H), :]
        p = jax.lax.dot_general(_bf(xg), _bf(wg), (((1,), (0,)), ((), ())),
                                preferred_element_type=jnp.float32)
        acc = p if acc is None else acc + p
    u_ref[...] = acc
    s1 = jnp.sum(acc, axis=0, keepdims=True)
    s2 = jnp.sum(acc * acc, axis=0, keepdims=True)
    st = jnp.concatenate([s1, s2], axis=0)

    @pl.when(pl.program_id(0) == 0)
    def _():
        s_ref[...] = st

    @pl.when(pl.program_id(0) != 0)
    def _():
        s_ref[...] = s_ref[...] + st


def matmul_stats(parts, w1, groups):
    m = parts[0].shape[-2]
    b = _pick_block(m)
    dout = w1.shape[1]
    in_specs = []
    for p in parts:
        if p.ndim == 3:
            in_specs.append(pl.BlockSpec((2, b, H), lambda i: (0, i, 0)))
        else:
            in_specs.append(pl.BlockSpec((b, H), lambda i: (i, 0)))
    in_specs.append(pl.BlockSpec(w1.shape, lambda i: (0, 0)))
    out_shape = [jax.ShapeDtypeStruct((m, dout), jnp.float32),
                 jax.ShapeDtypeStruct((2, dout), jnp.float32)]
    out_specs = [pl.BlockSpec((b, dout), lambda i: (i, 0)),
                 pl.BlockSpec((2, dout), lambda i: (0, 0))]
    return pl.pallas_call(
        functools.partial(_k1_body, groups, len(parts)),
        grid=(m // b,),
        in_specs=in_specs,
        out_specs=out_specs,
        out_shape=out_shape,
    )(*parts, w1)


def _bn_coeffs(s_ref, g_ref, b_ref, inv_m):
    mean = s_ref[0:1, :] * inv_m
    var = s_ref[1:2, :] * inv_m - mean * mean
    scale = g_ref[...] * jax.lax.rsqrt(var + EPS)
    shift = b_ref[...] - mean * scale
    return scale, shift


def _k2_body(inv_m, u_ref, s_ref, g_ref, b_ref, w_ref, v_ref, sv_ref):
    scale, shift = _bn_coeffs(s_ref, g_ref, b_ref, inv_m)
    h = jnp.maximum(u_ref[...] * scale + shift, 0.0)
    v = jax.lax.dot_general(_bf(h), _bf(w_ref[...]), (((1,), (0,)), ((), ())),
                            preferred_element_type=jnp.float32)
    v_ref[...] = v
    s1 = jnp.sum(v, axis=0, keepdims=True)
    s2 = jnp.sum(v * v, axis=0, keepdims=True)
    st = jnp.concatenate([s1, s2], axis=0)

    @pl.when(pl.program_id(0) == 0)
    def _():
        sv_ref[...] = st

    @pl.when(pl.program_id(0) != 0)
    def _():
        sv_ref[...] = sv_ref[...] + st


def bn_matmul_stats(u, s_u, g, bb, w2):
    m, din = u.shape
    dout = w2.shape[1]
    b = _pick_block(m)
    return pl.pallas_call(
        functools.partial(_k2_body, 1.0 / m),
        grid=(m // b,),
        in_specs=[pl.BlockSpec((b, din), lambda i: (i, 0)),
                  pl.BlockSpec((2, din), lambda i: (0, 0)),
                  pl.BlockSpec((1, din), lambda i: (0, 0)),
                  pl.BlockSpec((1, din), lambda i: (0, 0)),
                  pl.BlockSpec((din, dout), lambda i: (0, 0))],
        out_specs=[pl.BlockSpec((b, dout), lambda i: (i, 0)),
                   pl.BlockSpec((2, dout), lambda i: (0, 0))],
        out_shape=[jax.ShapeDtypeStruct((m, dout), jnp.float32),
                   jax.ShapeDtypeStruct((2, dout), jnp.float32)],
    )(u, s_u, g.reshape(1, din), bb.reshape(1, din), w2)


def _k3_body(inv_m, v_ref, s_ref, g_ref, b_ref, y_ref):
    scale, shift = _bn_coeffs(s_ref, g_ref, b_ref, inv_m)
    y_ref[...] = jnp.maximum(v_ref[...] * scale + shift, 0.0)


def bn_act(v, s_v, g, bb):
    m, d = v.shape
    b = _pick_block(m)
    return pl.pallas_call(
        functools.partial(_k3_body, 1.0 / m),
        grid=(m // b,),
        in_specs=[pl.BlockSpec((b, d), lambda i: (i, 0)),
                  pl.BlockSpec((2, d), lambda i: (0, 0)),
                  pl.BlockSpec((1, d), lambda i: (0, 0)),
                  pl.BlockSpec((1, d), lambda i: (0, 0))],
        out_specs=pl.BlockSpec((b, d), lambda i: (i, 0)),
        out_shape=jax.ShapeDtypeStruct((m, d), jnp.float32),
    )(v, s_v, g.reshape(1, d), bb.reshape(1, d))


def _k1f_body(inv_m, v0_ref, s0_ref, g0_ref, b0_ref,
              v2_ref, s2_ref, g2_ref, b2_ref, w_ref, u_ref, s_ref):
    sc0, sh0 = _bn_coeffs(s0_ref, g0_ref, b0_ref, inv_m)
    sc2, sh2 = _bn_coeffs(s2_ref, g2_ref, b2_ref, inv_m)
    x0 = jnp.maximum(v0_ref[...] * sc0 + sh0, 0.0)
    x2 = jnp.maximum(v2_ref[...] * sc2 + sh2, 0.0)
    u = (jax.lax.dot_general(_bf(x0), _bf(w_ref[pl.ds(0, H), :]),
                             (((1,), (0,)), ((), ())),
                             preferred_element_type=jnp.float32)
         + jax.lax.dot_general(_bf(x2), _bf(w_ref[pl.ds(H, H), :]),
                               (((1,), (0,)), ((), ())),
                               preferred_element_type=jnp.float32))
    u_ref[...] = u
    s1 = jnp.sum(u, axis=0, keepdims=True)
    s2 = jnp.sum(u * u, axis=0, keepdims=True)
    st = jnp.concatenate([s1, s2], axis=0)

    @pl.when(pl.program_id(0) == 0)
    def _():
        s_ref[...] = st

    @pl.when(pl.program_id(0) != 0)
    def _():
        s_ref[...] = s_ref[...] + st


def bn2_matmul_stats(v0, s0, g0, b0, v2, s2, g2, b2, w1):
    m = v0.shape[0]
    b = _pick_block(m)
    dout = w1.shape[1]
    sm = pl.BlockSpec((2, H), lambda i: (0, 0))
    gm = pl.BlockSpec((1, H), lambda i: (0, 0))
    return pl.pallas_call(
        functools.partial(_k1f_body, 1.0 / m),
        grid=(m // b,),
        in_specs=[pl.BlockSpec((b, H), lambda i: (i, 0)), sm, gm, gm,
                  pl.BlockSpec((b, H), lambda i: (i, 0)), sm, gm, gm,
                  pl.BlockSpec((2 * H, dout), lambda i: (0, 0))],
        out_specs=[pl.BlockSpec((b, dout), lambda i: (i, 0)),
                   pl.BlockSpec((2, dout), lambda i: (0, 0))],
        out_shape=[jax.ShapeDtypeStruct((m, dout), jnp.float32),
                   jax.ShapeDtypeStruct((2, dout), jnp.float32)],
    )(v0, s0, g0.reshape(1, H), b0.reshape(1, H),
      v2, s2, g2.reshape(1, H), b2.reshape(1, H), w1)


def _add2_body(x_ref, o_ref):
    o_ref[...] = x_ref[0] + x_ref[1]


def add_halves(x2):
    _, t, d = x2.shape
    b = _pick_block(t)
    return pl.pallas_call(
        _add2_body,
        grid=(t // b,),
        in_specs=[pl.BlockSpec((2, b, d), lambda i: (0, i, 0))],
        out_specs=pl.BlockSpec((b, d), lambda i: (i, 0)),
        out_shape=jax.ShapeDtypeStruct((t, d), jnp.float32),
    )(x2)


from jax.experimental.pallas import tpu_sc as plsc

_SC_MESH = plsc.VectorSubcoreMesh(core_axis_name="core",
                                  subcore_axis_name="subcore")


def _win(m, cap=128):
    for w in (128, 88, 80, 64, 40, 16):
        if w <= cap and m % w == 0:
            return w
    raise ValueError(m)


def sc_gather2(table, idx2):
    m = idx2.shape[1]
    w = _win(m)
    idx3 = jnp.stack([idx2[0].reshape(m // w, w), idx2[1].reshape(m // w, w)],
                     axis=1)

    @functools.partial(
        pl.kernel,
        out_type=jax.ShapeDtypeStruct((2, m, H), jnp.float32),
        mesh=_SC_MESH,
        scratch_types=[pltpu.SemaphoreType.DMA, pltpu.SemaphoreType.DMA],
    )
    def k(table_hbm, idx_hbm, out_hbm, sem0, sem1):
        def body(idx_vmem, out_vmem):
            c0 = pltpu.async_copy(table_hbm.at[idx_vmem.at[0, 0]],
                                  out_vmem.at[0], sem0)
            c1 = pltpu.async_copy(table_hbm.at[idx_vmem.at[0, 1]],
                                  out_vmem.at[1], sem1)
            c0.wait()
            c1.wait()

        pltpu.emit_pipeline(
            body,
            grid=(m // w,),
            in_specs=[pl.BlockSpec((1, 2, w), lambda i: (i, 0, 0))],
            out_specs=[pl.BlockSpec((2, w, H), lambda i: (0, i, 0))],
            core_axis_name=("core", "subcore"),
            dimension_semantics=(pltpu.PARALLEL,),
        )(idx_hbm, out_hbm)

    return k(table, idx3)


def sc_gather(table, idx):
    m = idx.shape[0]
    w = _win(m)
    idx3 = idx.reshape(m // w, 1, w)

    @functools.partial(
        pl.kernel,
        out_type=jax.ShapeDtypeStruct((m, H), jnp.float32),
        mesh=_SC_MESH,
        scratch_types=[pltpu.SemaphoreType.DMA],
    )
    def k(table_hbm, idx_hbm, out_hbm, sem):
        def body(idx_vmem, out_vmem):
            pltpu.async_copy(table_hbm.at[idx_vmem.at[0, 0]], out_vmem,
                             sem).wait()

        pltpu.emit_pipeline(
            body,
            grid=(m // w,),
            in_specs=[pl.BlockSpec((1, 1, w), lambda i: (i, 0, 0))],
            out_specs=[pl.BlockSpec((w, H), lambda i: (i, 0))],
            core_axis_name=("core", "subcore"),
            dimension_semantics=(pltpu.PARALLEL,),
        )(idx_hbm, out_hbm)

    return k(table, idx3)


def sc_scatter_add(rows, idx, t):
    kn, m = idx.shape
    w = _win(m)
    idx3 = jnp.stack([idx[j].reshape(m // w, w) for j in range(kn)], axis=1)
    cz = 80
    nch = t // cz
    per_tile = -(-nch // 16)
    zrs = jnp.zeros((cz, H), jnp.float32)

    @functools.partial(
        pl.kernel,
        out_type=jax.ShapeDtypeStruct((2, t, H), jnp.float32),
        mesh=_SC_MESH,
        scratch_types=[pltpu.VMEM_SHARED((t, H), jnp.float32)],
    )
    def k(rows_hbm, idx_hbm, z_hbm, out_hbm, acc):
        cid = lax.axis_index("core")
        sid = lax.axis_index("subcore")

        @pl.loop(0, per_tile)
        def _(i):
            ch = i * 16 + sid

            @pl.when(ch < nch)
            def _():
                off = pl.multiple_of(ch * cz, 8)
                pltpu.sync_copy(z_hbm, acc.at[pl.ds(off, cz)])

        plsc.subcore_barrier()

        def body(rows_vmem, idx_vmem):
            for j in range(kn):
                pltpu.sync_copy(rows_vmem, acc.at[idx_vmem.at[0, j]],
                                add=True)

        pltpu.emit_pipeline(
            body,
            grid=(m // w,),
            in_specs=[pl.BlockSpec((w, H), lambda i: (i, 0)),
                      pl.BlockSpec((1, kn, w), lambda i: (i, 0, 0))],
            out_specs=[],
            core_axis_name=("core", "subcore"),
            dimension_semantics=(pltpu.PARALLEL,),
        )(rows_hbm, idx_hbm)

        plsc.subcore_barrier()

        @pl.loop(0, per_tile)
        def _(i):
            ch = i * 16 + sid

            @pl.when(ch < nch)
            def _():
                off = pl.multiple_of(ch * cz, 8)
                pltpu.sync_copy(acc.at[pl.ds(off, cz)],
                                out_hbm.at[cid].at[pl.ds(off, cz)])

    return k(rows, idx3, zrs)


def sc_segment_sum_c(rows, idx, c):
    m = idx.shape[0]
    w = 80
    assert m % w == 0 and c % 2 == 0
    half = c // 2
    pad = half + 80
    idx3 = idx.reshape(m // w, 1, w)
    cz = 80
    nzch = pad // cz
    noch = half // cz
    zrs = jnp.zeros((cz, H), jnp.float32)

    @functools.partial(
        pl.kernel,
        out_type=jax.ShapeDtypeStruct((2, half, H), jnp.float32),
        mesh=_SC_MESH,
        scratch_types=[pltpu.VMEM_SHARED((pad, H), jnp.float32),
                       pltpu.VMEM((1, w), jnp.int32)],
    )
    def k(rows_hbm, idx_hbm, z_hbm, out_hbm, acc, sidx):
        cid = lax.axis_index("core")
        sid = lax.axis_index("subcore")
        lo = cid * half

        @pl.loop(0, -(-nzch // 16))
        def _(i):
            ch = i * 16 + sid

            @pl.when(ch < nzch)
            def _():
                off = pl.multiple_of(ch * cz, 8)
                pltpu.sync_copy(z_hbm, acc.at[pl.ds(off, cz)])

        plsc.subcore_barrier()

        def body(rows_vmem, idx_vmem):
            for kk in range(w // 16):
                v = idx_vmem[0, 0, pl.ds(kk * 16, 16)]
                ok = (v >= lo) & (v < lo + half)
                sidx[0, pl.ds(kk * 16, 16)] = jnp.where(ok, v - lo, half)
            pltpu.sync_copy(rows_vmem, acc.at[sidx.at[0]], add=True)

        pltpu.emit_pipeline(
            body,
            grid=(m // w,),
            in_specs=[pl.BlockSpec((w, H), lambda i: (i, 0)),
                      pl.BlockSpec((1, 1, w), lambda i: (i, 0, 0))],
            out_specs=[],
            core_axis_name="subcore",
            dimension_semantics=(pltpu.PARALLEL,),
        )(rows_hbm, idx_hbm)

        plsc.subcore_barrier()

        @pl.loop(0, -(-noch // 16))
        def _(i):
            ch = i * 16 + sid

            @pl.when(ch < noch)
            def _():
                off = pl.multiple_of(ch * cz, 8)
                pltpu.sync_copy(acc.at[pl.ds(off, cz)],
                                out_hbm.at[cid].at[pl.ds(off, cz)])

    return k(rows, idx3, zrs).reshape(c, H)


def _mlp(parts, groups, p):
    u, su = matmul_stats(parts, p["W1"], groups)
    v, sv = bn_matmul_stats(u, su, p["g1"], p["b1"], p["W2"])
    return bn_act(v, sv, p["g2"], p["b2"])


def kernel(node_rep, edge_rep, cycle_rep, edge_index, cycle_node_ids,
           cycle_ids, node_mlp, edge_mlp0, cycle_mlp, edge_mlpc, edge_mlpt):
    n = node_rep.shape[0]
    c = 16000
    src, dst = edge_index[0], edge_index[1]

    n2e = sc_gather2(node_rep, edge_index)
    u0, su0 = matmul_stats([edge_rep, n2e], edge_mlp0["W1"], [[0], [1]])
    v0, sv0 = bn_matmul_stats(u0, su0, edge_mlp0["g1"], edge_mlp0["b1"],
                              edge_mlp0["W2"])
    edge_out0 = bn_act(v0, sv0, edge_mlp0["g2"], edge_mlp0["b2"])

    e_at_n = add_halves(sc_scatter_add(edge_rep, edge_index, n))
    per_row = sc_gather(e_at_n, cycle_node_ids)
    cyc_sum = sc_segment_sum_c(per_row, cycle_ids, c)
    gcyc = sc_gather(cyc_sum, cycle_ids)
    cycle_out = _mlp([cycle_rep, per_row, gcyc], [[0], [1], [2]], cycle_mlp)

    c_at_n = add_halves(sc_scatter_add(cycle_out,
                                       cycle_node_ids.reshape(1, -1), n))
    c2e = sc_gather2(c_at_n, edge_index)

    e2n = sc_scatter_add(edge_out0, edge_index, n)
    node_out = _mlp([node_rep, e2n], [[0], [1]], node_mlp)

    uc, suc = matmul_stats([edge_rep, c2e], edge_mlpc["W1"], [[0], [1]])
    vc, svc = bn_matmul_stats(uc, suc, edge_mlpc["g1"], edge_mlpc["b1"],
                              edge_mlpc["W2"])

    ut, sut = bn2_matmul_stats(v0, sv0, edge_mlp0["g2"], edge_mlp0["b2"],
                               vc, svc, edge_mlpc["g2"], edge_mlpc["b2"],
                               edge_mlpt["W1"])
    vt, svt = bn_matmul_stats(ut, sut, edge_mlpt["g1"], edge_mlpt["b1"],
                              edge_mlpt["W2"])
    edge_out = bn_act(vt, svt, edge_mlpt["g2"], edge_mlpt["b2"])

    return (node_out, edge_out, cycle_out)


# speedup vs baseline: 3.2543x; 1.0276x over previous
import functools

import jax
import jax.numpy as jnp
from jax import lax
from jax.experimental import pallas as pl
from jax.experimental.pallas import tpu as pltpu

H = 128
EPS = 1e-5


def _pick_block(m):
    for b in (8000, 4000, 2000, 1000, 500, 250, 125, 100, 40, 8):
        if m % b == 0:
            return b
    return m


def _bf(x):
    return x


def _k1_body(groups, nparts, *refs):
    part_refs = refs[:nparts]
    w_ref = refs[nparts]
    u_ref, s_ref = refs[nparts + 1], refs[nparts + 2]
    acc = None
    for gi, members in enumerate(groups):
        xg = None
        for pix in members:
            r = part_refs[pix]
            if len(r.shape) == 3:
                xv = r[0] + r[1]
            else:
                xv = r[...]
            xg = xv if xg is None else xg + xv
        wg = w_ref[pl.ds(gi * H, H), :]
        p = jax.lax.dot_general(_bf(xg), _bf(wg), (((1,), (0,)), ((), ())),
                                preferred_element_type=jnp.float32)
        acc = p if acc is None else acc + p
    u_ref[...] = acc
    s1 = jnp.sum(acc, axis=0, keepdims=True)
    s2 = jnp.sum(acc * acc, axis=0, keepdims=True)
    st = jnp.concatenate([s1, s2], axis=0)

    @pl.when(pl.program_id(0) == 0)
    def _():
        s_ref[...] = st

    @pl.when(pl.program_id(0) != 0)
    def _():
        s_ref[...] = s_ref[...] + st


def matmul_stats(parts, w1, groups):
    m = parts[0].shape[-2]
    b = _pick_block(m)
    dout = w1.shape[1]
    in_specs = []
    for p in parts:
        if p.ndim == 3:
            in_specs.append(pl.BlockSpec((2, b, H), lambda i: (0, i, 0)))
        else:
            in_specs.append(pl.BlockSpec((b, H), lambda i: (i, 0)))
    in_specs.append(pl.BlockSpec(w1.shape, lambda i: (0, 0)))
    out_shape = [jax.ShapeDtypeStruct((m, dout), jnp.float32),
                 jax.ShapeDtypeStruct((2, dout), jnp.float32)]
    out_specs = [pl.BlockSpec((b, dout), lambda i: (i, 0)),
                 pl.BlockSpec((2, dout), lambda i: (0, 0))]
    return pl.pallas_call(
        functools.partial(_k1_body, groups, len(parts)),
        grid=(m // b,),
        in_specs=in_specs,
        out_specs=out_specs,
        out_shape=out_shape,
    )(*parts, w1)


def _bn_coeffs(s_ref, g_ref, b_ref, inv_m):
    mean = s_ref[0:1, :] * inv_m
    var = s_ref[1:2, :] * inv_m - mean * mean
    scale = g_ref[...] * jax.lax.rsqrt(var + EPS)
    shift = b_ref[...] - mean * scale
    return scale, shift


def _k2_body(inv_m, u_ref, s_ref, g_ref, b_ref, w_ref, v_ref, sv_ref):
    scale, shift = _bn_coeffs(s_ref, g_ref, b_ref, inv_m)
    h = jnp.maximum(u_ref[...] * scale + shift, 0.0)
    v = jax.lax.dot_general(_bf(h), _bf(w_ref[...]), (((1,), (0,)), ((), ())),
                            preferred_element_type=jnp.float32)
    v_ref[...] = v
    s1 = jnp.sum(v, axis=0, keepdims=True)
    s2 = jnp.sum(v * v, axis=0, keepdims=True)
    st = jnp.concatenate([s1, s2], axis=0)

    @pl.when(pl.program_id(0) == 0)
    def _():
        sv_ref[...] = st

    @pl.when(pl.program_id(0) != 0)
    def _():
        sv_ref[...] = sv_ref[...] + st


def bn_matmul_stats(u, s_u, g, bb, w2):
    m, din = u.shape
    dout = w2.shape[1]
    b = _pick_block(m)
    return pl.pallas_call(
        functools.partial(_k2_body, 1.0 / m),
        grid=(m // b,),
        in_specs=[pl.BlockSpec((b, din), lambda i: (i, 0)),
                  pl.BlockSpec((2, din), lambda i: (0, 0)),
                  pl.BlockSpec((1, din), lambda i: (0, 0)),
                  pl.BlockSpec((1, din), lambda i: (0, 0)),
                  pl.BlockSpec((din, dout), lambda i: (0, 0))],
        out_specs=[pl.BlockSpec((b, dout), lambda i: (i, 0)),
                   pl.BlockSpec((2, dout), lambda i: (0, 0))],
        out_shape=[jax.ShapeDtypeStruct((m, dout), jnp.float32),
                   jax.ShapeDtypeStruct((2, dout), jnp.float32)],
    )(u, s_u, g.reshape(1, din), bb.reshape(1, din), w2)


def _k3_body(inv_m, v_ref, s_ref, g_ref, b_ref, y_ref):
    scale, shift = _bn_coeffs(s_ref, g_ref, b_ref, inv_m)
    y_ref[...] = jnp.maximum(v_ref[...] * scale + shift, 0.0)


def bn_act(v, s_v, g, bb):
    m, d = v.shape
    b = _pick_block(m)
    return pl.pallas_call(
        functools.partial(_k3_body, 1.0 / m),
        grid=(m // b,),
        in_specs=[pl.BlockSpec((b, d), lambda i: (i, 0)),
                  pl.BlockSpec((2, d), lambda i: (0, 0)),
                  pl.BlockSpec((1, d), lambda i: (0, 0)),
                  pl.BlockSpec((1, d), lambda i: (0, 0))],
        out_specs=pl.BlockSpec((b, d), lambda i: (i, 0)),
        out_shape=jax.ShapeDtypeStruct((m, d), jnp.float32),
    )(v, s_v, g.reshape(1, d), bb.reshape(1, d))


def _k1f_body(inv_m, v0_ref, s0_ref, g0_ref, b0_ref,
              v2_ref, s2_ref, g2_ref, b2_ref, w_ref, u_ref, s_ref):
    sc0, sh0 = _bn_coeffs(s0_ref, g0_ref, b0_ref, inv_m)
    sc2, sh2 = _bn_coeffs(s2_ref, g2_ref, b2_ref, inv_m)
    x0 = jnp.maximum(v0_ref[...] * sc0 + sh0, 0.0)
    x2 = jnp.maximum(v2_ref[...] * sc2 + sh2, 0.0)
    u = (jax.lax.dot_general(_bf(x0), _bf(w_ref[pl.ds(0, H), :]),
                             (((1,), (0,)), ((), ())),
                             preferred_element_type=jnp.float32)
         + jax.lax.dot_general(_bf(x2), _bf(w_ref[pl.ds(H, H), :]),
                               (((1,), (0,)), ((), ())),
                               preferred_element_type=jnp.float32))
    u_ref[...] = u
    s1 = jnp.sum(u, axis=0, keepdims=True)
    s2 = jnp.sum(u * u, axis=0, keepdims=True)
    st = jnp.concatenate([s1, s2], axis=0)

    @pl.when(pl.program_id(0) == 0)
    def _():
        s_ref[...] = st

    @pl.when(pl.program_id(0) != 0)
    def _():
        s_ref[...] = s_ref[...] + st


def bn2_matmul_stats(v0, s0, g0, b0, v2, s2, g2, b2, w1):
    m = v0.shape[0]
    b = _pick_block(m)
    dout = w1.shape[1]
    sm = pl.BlockSpec((2, H), lambda i: (0, 0))
    gm = pl.BlockSpec((1, H), lambda i: (0, 0))
    return pl.pallas_call(
        functools.partial(_k1f_body, 1.0 / m),
        grid=(m // b,),
        in_specs=[pl.BlockSpec((b, H), lambda i: (i, 0)), sm, gm, gm,
                  pl.BlockSpec((b, H), lambda i: (i, 0)), sm, gm, gm,
                  pl.BlockSpec((2 * H, dout), lambda i: (0, 0))],
        out_specs=[pl.BlockSpec((b, dout), lambda i: (i, 0)),
                   pl.BlockSpec((2, dout), lambda i: (0, 0))],
        out_shape=[jax.ShapeDtypeStruct((m, dout), jnp.float32),
                   jax.ShapeDtypeStruct((2, dout), jnp.float32)],
    )(v0, s0, g0.reshape(1, H), b0.reshape(1, H),
      v2, s2, g2.reshape(1, H), b2.reshape(1, H), w1)


def _add2_body(x_ref, o_ref):
    o_ref[...] = x_ref[0] + x_ref[1]


def add_halves(x2):
    _, t, d = x2.shape
    b = _pick_block(t)
    return pl.pallas_call(
        _add2_body,
        grid=(t // b,),
        in_specs=[pl.BlockSpec((2, b, d), lambda i: (0, i, 0))],
        out_specs=pl.BlockSpec((b, d), lambda i: (i, 0)),
        out_shape=jax.ShapeDtypeStruct((t, d), jnp.float32),
    )(x2)


from jax.experimental.pallas import tpu_sc as plsc

_SC_MESH = plsc.VectorSubcoreMesh(core_axis_name="core",
                                  subcore_axis_name="subcore")


def _win(m, cap=128):
    for w in (128, 88, 80, 64, 40, 16):
        if w <= cap and m % w == 0:
            return w
    raise ValueError(m)


def sc_gather2(table, idx2):
    m = idx2.shape[1]
    w = _win(m)
    idx3 = jnp.stack([idx2[0].reshape(m // w, w), idx2[1].reshape(m // w, w)],
                     axis=1)

    @functools.partial(
        pl.kernel,
        out_type=jax.ShapeDtypeStruct((2, m, H), jnp.float32),
        mesh=_SC_MESH,
        scratch_types=[pltpu.SemaphoreType.DMA, pltpu.SemaphoreType.DMA],
    )
    def k(table_hbm, idx_hbm, out_hbm, sem0, sem1):
        def body(idx_vmem, out_vmem):
            c0 = pltpu.async_copy(table_hbm.at[idx_vmem.at[0, 0]],
                                  out_vmem.at[0], sem0)
            c1 = pltpu.async_copy(table_hbm.at[idx_vmem.at[0, 1]],
                                  out_vmem.at[1], sem1)
            c0.wait()
            c1.wait()

        pltpu.emit_pipeline(
            body,
            grid=(m // w,),
            in_specs=[pl.BlockSpec((1, 2, w), lambda i: (i, 0, 0))],
            out_specs=[pl.BlockSpec((2, w, H), lambda i: (0, i, 0))],
            core_axis_name=("core", "subcore"),
            dimension_semantics=(pltpu.PARALLEL,),
        )(idx_hbm, out_hbm)

    return k(table, idx3)


def sc_gather(table, idx):
    m = idx.shape[0]
    w = _win(m)
    idx3 = idx.reshape(m // w, 1, w)

    @functools.partial(
        pl.kernel,
        out_type=jax.ShapeDtypeStruct((m, H), jnp.float32),
        mesh=_SC_MESH,
        scratch_types=[pltpu.SemaphoreType.DMA],
    )
    def k(table_hbm, idx_hbm, out_hbm, sem):
        def body(idx_vmem, out_vmem):
            pltpu.async_copy(table_hbm.at[idx_vmem.at[0, 0]], out_vmem,
                             sem).wait()

        pltpu.emit_pipeline(
            body,
            grid=(m // w,),
            in_specs=[pl.BlockSpec((1, 1, w), lambda i: (i, 0, 0))],
            out_specs=[pl.BlockSpec((w, H), lambda i: (i, 0))],
            core_axis_name=("core", "subcore"),
            dimension_semantics=(pltpu.PARALLEL,),
        )(idx_hbm, out_hbm)

    return k(table, idx3)


def sc_scatter_add(rows, idx, t):
    kn, m = idx.shape
    w = _win(m)
    idx3 = jnp.stack([idx[j].reshape(m // w, w) for j in range(kn)], axis=1)
    cz = 80
    nch = t // cz
    per_tile = -(-nch // 16)
    zrs = jnp.zeros((cz, H), jnp.float32)

    @functools.partial(
        pl.kernel,
        out_type=jax.ShapeDtypeStruct((2, t, H), jnp.float32),
        mesh=_SC_MESH,
        scratch_types=[pltpu.VMEM_SHARED((t, H), jnp.float32)],
    )
    def k(rows_hbm, idx_hbm, z_hbm, out_hbm, acc):
        cid = lax.axis_index("core")
        sid = lax.axis_index("subcore")

        @pl.loop(0, per_tile)
        def _(i):
            ch = i * 16 + sid

            @pl.when(ch < nch)
            def _():
                off = pl.multiple_of(ch * cz, 8)
                pltpu.sync_copy(z_hbm, acc.at[pl.ds(off, cz)])

        plsc.subcore_barrier()

        def body(rows_vmem, idx_vmem):
            for j in range(kn):
                pltpu.sync_copy(rows_vmem, acc.at[idx_vmem.at[0, j]],
                                add=True)

        pltpu.emit_pipeline(
            body,
            grid=(m // w,),
            in_specs=[pl.BlockSpec((w, H), lambda i: (i, 0)),
                      pl.BlockSpec((1, kn, w), lambda i: (i, 0, 0))],
            out_specs=[],
            core_axis_name=("core", "subcore"),
            dimension_semantics=(pltpu.PARALLEL,),
        )(rows_hbm, idx_hbm)

        plsc.subcore_barrier()

        @pl.loop(0, per_tile)
        def _(i):
            ch = i * 16 + sid

            @pl.when(ch < nch)
            def _():
                off = pl.multiple_of(ch * cz, 8)
                pltpu.sync_copy(acc.at[pl.ds(off, cz)],
                                out_hbm.at[cid].at[pl.ds(off, cz)])

    return k(rows, idx3, zrs)


def sc_segment_sum_c(rows, idx, c):
    m = idx.shape[0]
    w = 80
    assert m % w == 0 and c % 2 == 0
    half = c // 2
    pad = half + 80
    idx3 = idx.reshape(m // w, 1, w)
    cz = 80
    nzch = pad // cz
    noch = half // cz
    zrs = jnp.zeros((cz, H), jnp.float32)

    @functools.partial(
        pl.kernel,
        out_type=jax.ShapeDtypeStruct((2, half, H), jnp.float32),
        mesh=_SC_MESH,
        scratch_types=[pltpu.VMEM_SHARED((pad, H), jnp.float32),
                       pltpu.VMEM((1, w), jnp.int32)],
    )
    def k(rows_hbm, idx_hbm, z_hbm, out_hbm, acc, sidx):
        cid = lax.axis_index("core")
        sid = lax.axis_index("subcore")
        lo = cid * half

        @pl.loop(0, -(-nzch // 16))
        def _(i):
            ch = i * 16 + sid

            @pl.when(ch < nzch)
            def _():
                off = pl.multiple_of(ch * cz, 8)
                pltpu.sync_copy(z_hbm, acc.at[pl.ds(off, cz)])

        plsc.subcore_barrier()

        def body(rows_vmem, idx_vmem):
            for kk in range(w // 16):
                v = idx_vmem[0, 0, pl.ds(kk * 16, 16)]
                ok = (v >= lo) & (v < lo + half)
                sidx[0, pl.ds(kk * 16, 16)] = jnp.where(ok, v - lo, half)
            pltpu.sync_copy(rows_vmem, acc.at[sidx.at[0]], add=True)

        pltpu.emit_pipeline(
            body,
            grid=(m // w,),
            in_specs=[pl.BlockSpec((w, H), lambda i: (i, 0)),
                      pl.BlockSpec((1, 1, w), lambda i: (i, 0, 0))],
            out_specs=[],
            core_axis_name="subcore",
            dimension_semantics=(pltpu.PARALLEL,),
        )(rows_hbm, idx_hbm)

        plsc.subcore_barrier()

        @pl.loop(0, -(-noch // 16))
        def _(i):
            ch = i * 16 + sid

            @pl.when(ch < noch)
            def _():
                off = pl.multiple_of(ch * cz, 8)
                pltpu.sync_copy(acc.at[pl.ds(off, cz)],
                                out_hbm.at[cid].at[pl.ds(off, cz)])

    return k(rows, idx3, zrs).reshape(c, H)


def _mlp(parts, groups, p):
    u, su = matmul_stats(parts, p["W1"], groups)
    v, sv = bn_matmul_stats(u, su, p["g1"], p["b1"], p["W2"])
    return bn_act(v, sv, p["g2"], p["b2"])


def kernel(node_rep, edge_rep, cycle_rep, edge_index, cycle_node_ids,
           cycle_ids, node_mlp, edge_mlp0, cycle_mlp, edge_mlpc, edge_mlpt):
    n = node_rep.shape[0]
    c = 16000
    src, dst = edge_index[0], edge_index[1]

    n2e = sc_gather2(node_rep, edge_index)
    u0, su0 = matmul_stats([edge_rep, n2e], edge_mlp0["W1"], [[0], [1]])
    v0, sv0 = bn_matmul_stats(u0, su0, edge_mlp0["g1"], edge_mlp0["b1"],
                              edge_mlp0["W2"])
    edge_out0 = bn_act(v0, sv0, edge_mlp0["g2"], edge_mlp0["b2"])

    e_at_n = add_halves(sc_scatter_add(edge_rep, edge_index, n))
    per_row = sc_gather(e_at_n, cycle_node_ids)
    cyc_sum = sc_segment_sum_c(per_row, cycle_ids, c)
    gcyc = sc_gather(cyc_sum, cycle_ids)
    cycle_out = _mlp([cycle_rep, per_row, gcyc], [[0], [1], [2]], cycle_mlp)

    c_at_n = add_halves(sc_scatter_add(cycle_out,
                                       cycle_node_ids.reshape(1, -1), n))
    c2e = sc_gather2(c_at_n, edge_index)

    e2n = sc_scatter_add(edge_out0, edge_index, n)
    node_out = _mlp([node_rep, e2n], [[0], [1]], node_mlp)

    uc, suc = matmul_stats([edge_rep, c2e], edge_mlpc["W1"], [[0], [1]])
    vc, svc = bn_matmul_stats(uc, suc, edge_mlpc["g1"], edge_mlpc["b1"],
                              edge_mlpc["W2"])

    ut, sut = bn2_matmul_stats(v0, sv0, edge_mlp0["g2"], edge_mlp0["b2"],
                               vc, svc, edge_mlpc["g2"], edge_mlpc["b2"],
                               edge_mlpt["W1"])
    vt, svt = bn_matmul_stats(ut, sut, edge_mlpt["g1"], edge_mlpt["b1"],
                              edge_mlpt["W2"])
    edge_out = bn_act(vt, svt, edge_mlpt["g2"], edge_mlpt["b2"])

    return (node_out, edge_out, cycle_out)


# speedup vs baseline: 3.2568x; 1.0008x over previous
import functools

import jax
import jax.numpy as jnp
from jax import lax
from jax.experimental import pallas as pl
from jax.experimental.pallas import tpu as pltpu

H = 128
EPS = 1e-5


def _pick_block(m):
    for b in (8000, 4000, 2000, 1000, 500, 250, 125, 100, 40, 8):
        if m % b == 0:
            return b
    return m


def _bf(x):
    return x


def _k1_body(groups, nparts, *refs):
    part_refs = refs[:nparts]
    w_ref = refs[nparts]
    u_ref, s_ref = refs[nparts + 1], refs[nparts + 2]
    acc = None
    for gi, members in enumerate(groups):
        xg = None
        for pix in members:
            r = part_refs[pix]
            if len(r.shape) == 3:
                xv = r[0] + r[1]
            else:
                xv = r[...]
            xg = xv if xg is None else xg + xv
        wg = w_ref[pl.ds(gi * H, H), :]
        p = jax.lax.dot_general(_bf(xg), _bf(wg), (((1,), (0,)), ((), ())),
                                preferred_element_type=jnp.float32)
        acc = p if acc is None else acc + p
    u_ref[...] = acc
    s1 = jnp.sum(acc, axis=0, keepdims=True)
    s2 = jnp.sum(acc * acc, axis=0, keepdims=True)
    st = jnp.concatenate([s1, s2], axis=0)

    @pl.when(pl.program_id(0) == 0)
    def _():
        s_ref[...] = st

    @pl.when(pl.program_id(0) != 0)
    def _():
        s_ref[...] = s_ref[...] + st


def matmul_stats(parts, w1, groups):
    m = parts[0].shape[-2]
    b = _pick_block(m)
    dout = w1.shape[1]
    in_specs = []
    for p in parts:
        if p.ndim == 3:
            in_specs.append(pl.BlockSpec((2, b, H), lambda i: (0, i, 0)))
        else:
            in_specs.append(pl.BlockSpec((b, H), lambda i: (i, 0)))
    in_specs.append(pl.BlockSpec(w1.shape, lambda i: (0, 0)))
    out_shape = [jax.ShapeDtypeStruct((m, dout), jnp.float32),
                 jax.ShapeDtypeStruct((2, dout), jnp.float32)]
    out_specs = [pl.BlockSpec((b, dout), lambda i: (i, 0)),
                 pl.BlockSpec((2, dout), lambda i: (0, 0))]
    return pl.pallas_call(
        functools.partial(_k1_body, groups, len(parts)),
        grid=(m // b,),
        in_specs=in_specs,
        out_specs=out_specs,
        out_shape=out_shape,
    )(*parts, w1)


def _bn_coeffs(s_ref, g_ref, b_ref, inv_m):
    mean = s_ref[0:1, :] * inv_m
    var = s_ref[1:2, :] * inv_m - mean * mean
    scale = g_ref[...] * jax.lax.rsqrt(var + EPS)
    shift = b_ref[...] - mean * scale
    return scale, shift


def _k2_body(inv_m, u_ref, s_ref, g_ref, b_ref, w_ref, v_ref, sv_ref):
    scale, shift = _bn_coeffs(s_ref, g_ref, b_ref, inv_m)
    h = jnp.maximum(u_ref[...] * scale + shift, 0.0)
    v = jax.lax.dot_general(_bf(h), _bf(w_ref[...]), (((1,), (0,)), ((), ())),
                            preferred_element_type=jnp.float32)
    v_ref[...] = v
    s1 = jnp.sum(v, axis=0, keepdims=True)
    s2 = jnp.sum(v * v, axis=0, keepdims=True)
    st = jnp.concatenate([s1, s2], axis=0)

    @pl.when(pl.program_id(0) == 0)
    def _():
        sv_ref[...] = st

    @pl.when(pl.program_id(0) != 0)
    def _():
        sv_ref[...] = sv_ref[...] + st


def bn_matmul_stats(u, s_u, g, bb, w2):
    m, din = u.shape
    dout = w2.shape[1]
    b = _pick_block(m)
    return pl.pallas_call(
        functools.partial(_k2_body, 1.0 / m),
        grid=(m // b,),
        in_specs=[pl.BlockSpec((b, din), lambda i: (i, 0)),
                  pl.BlockSpec((2, din), lambda i: (0, 0)),
                  pl.BlockSpec((1, din), lambda i: (0, 0)),
                  pl.BlockSpec((1, din), lambda i: (0, 0)),
                  pl.BlockSpec((din, dout), lambda i: (0, 0))],
        out_specs=[pl.BlockSpec((b, dout), lambda i: (i, 0)),
                   pl.BlockSpec((2, dout), lambda i: (0, 0))],
        out_shape=[jax.ShapeDtypeStruct((m, dout), jnp.float32),
                   jax.ShapeDtypeStruct((2, dout), jnp.float32)],
    )(u, s_u, g.reshape(1, din), bb.reshape(1, din), w2)


def _k3_body(inv_m, v_ref, s_ref, g_ref, b_ref, y_ref):
    scale, shift = _bn_coeffs(s_ref, g_ref, b_ref, inv_m)
    y_ref[...] = jnp.maximum(v_ref[...] * scale + shift, 0.0)


def bn_act(v, s_v, g, bb):
    m, d = v.shape
    b = _pick_block(m)
    return pl.pallas_call(
        functools.partial(_k3_body, 1.0 / m),
        grid=(m // b,),
        in_specs=[pl.BlockSpec((b, d), lambda i: (i, 0)),
                  pl.BlockSpec((2, d), lambda i: (0, 0)),
                  pl.BlockSpec((1, d), lambda i: (0, 0)),
                  pl.BlockSpec((1, d), lambda i: (0, 0))],
        out_specs=pl.BlockSpec((b, d), lambda i: (i, 0)),
        out_shape=jax.ShapeDtypeStruct((m, d), jnp.float32),
    )(v, s_v, g.reshape(1, d), bb.reshape(1, d))


def _k1f_body(inv_m, v0_ref, s0_ref, g0_ref, b0_ref,
              v2_ref, s2_ref, g2_ref, b2_ref, w_ref, u_ref, s_ref):
    sc0, sh0 = _bn_coeffs(s0_ref, g0_ref, b0_ref, inv_m)
    sc2, sh2 = _bn_coeffs(s2_ref, g2_ref, b2_ref, inv_m)
    x0 = jnp.maximum(v0_ref[...] * sc0 + sh0, 0.0)
    x2 = jnp.maximum(v2_ref[...] * sc2 + sh2, 0.0)
    u = (jax.lax.dot_general(_bf(x0), _bf(w_ref[pl.ds(0, H), :]),
                             (((1,), (0,)), ((), ())),
                             preferred_element_type=jnp.float32)
         + jax.lax.dot_general(_bf(x2), _bf(w_ref[pl.ds(H, H), :]),
                               (((1,), (0,)), ((), ())),
                               preferred_element_type=jnp.float32))
    u_ref[...] = u
    s1 = jnp.sum(u, axis=0, keepdims=True)
    s2 = jnp.sum(u * u, axis=0, keepdims=True)
    st = jnp.concatenate([s1, s2], axis=0)

    @pl.when(pl.program_id(0) == 0)
    def _():
        s_ref[...] = st

    @pl.when(pl.program_id(0) != 0)
    def _():
        s_ref[...] = s_ref[...] + st


def bn2_matmul_stats(v0, s0, g0, b0, v2, s2, g2, b2, w1):
    m = v0.shape[0]
    b = _pick_block(m)
    dout = w1.shape[1]
    sm = pl.BlockSpec((2, H), lambda i: (0, 0))
    gm = pl.BlockSpec((1, H), lambda i: (0, 0))
    return pl.pallas_call(
        functools.partial(_k1f_body, 1.0 / m),
        grid=(m // b,),
        in_specs=[pl.BlockSpec((b, H), lambda i: (i, 0)), sm, gm, gm,
                  pl.BlockSpec((b, H), lambda i: (i, 0)), sm, gm, gm,
                  pl.BlockSpec((2 * H, dout), lambda i: (0, 0))],
        out_specs=[pl.BlockSpec((b, dout), lambda i: (i, 0)),
                   pl.BlockSpec((2, dout), lambda i: (0, 0))],
        out_shape=[jax.ShapeDtypeStruct((m, dout), jnp.float32),
                   jax.ShapeDtypeStruct((2, dout), jnp.float32)],
    )(v0, s0, g0.reshape(1, H), b0.reshape(1, H),
      v2, s2, g2.reshape(1, H), b2.reshape(1, H), w1)


def _add2_body(x_ref, o_ref):
    o_ref[...] = x_ref[0] + x_ref[1]


def add_halves(x2):
    _, t, d = x2.shape
    b = _pick_block(t)
    return pl.pallas_call(
        _add2_body,
        grid=(t // b,),
        in_specs=[pl.BlockSpec((2, b, d), lambda i: (0, i, 0))],
        out_specs=pl.BlockSpec((b, d), lambda i: (i, 0)),
        out_shape=jax.ShapeDtypeStruct((t, d), jnp.float32),
    )(x2)


from jax.experimental.pallas import tpu_sc as plsc

_SC_MESH = plsc.VectorSubcoreMesh(core_axis_name="core",
                                  subcore_axis_name="subcore")


def _win(m, cap=128):
    for w in (128, 88, 80, 64, 40, 16):
        if w <= cap and m % w == 0:
            return w
    raise ValueError(m)


def sc_gather2(table, idx2):
    m = idx2.shape[1]
    w = _win(m)
    idx3 = jnp.stack([idx2[0].reshape(m // w, w), idx2[1].reshape(m // w, w)],
                     axis=1)

    @functools.partial(
        pl.kernel,
        out_type=jax.ShapeDtypeStruct((2, m, H), jnp.float32),
        mesh=_SC_MESH,
        scratch_types=[pltpu.SemaphoreType.DMA, pltpu.SemaphoreType.DMA],
    )
    def k(table_hbm, idx_hbm, out_hbm, sem0, sem1):
        def body(idx_vmem, out_vmem):
            c0 = pltpu.async_copy(table_hbm.at[idx_vmem.at[0, 0]],
                                  out_vmem.at[0], sem0)
            c1 = pltpu.async_copy(table_hbm.at[idx_vmem.at[0, 1]],
                                  out_vmem.at[1], sem1)
            c0.wait()
            c1.wait()

        pltpu.emit_pipeline(
            body,
            grid=(m // w,),
            in_specs=[pl.BlockSpec((1, 2, w), lambda i: (i, 0, 0))],
            out_specs=[pl.BlockSpec((2, w, H), lambda i: (0, i, 0))],
            core_axis_name=("core", "subcore"),
            dimension_semantics=(pltpu.PARALLEL,),
        )(idx_hbm, out_hbm)

    return k(table, idx3)


def sc_gather(table, idx):
    m = idx.shape[0]
    w = _win(m)
    idx3 = idx.reshape(m // w, 1, w)

    @functools.partial(
        pl.kernel,
        out_type=jax.ShapeDtypeStruct((m, H), jnp.float32),
        mesh=_SC_MESH,
        scratch_types=[pltpu.SemaphoreType.DMA],
    )
    def k(table_hbm, idx_hbm, out_hbm, sem):
        def body(idx_vmem, out_vmem):
            pltpu.async_copy(table_hbm.at[idx_vmem.at[0, 0]], out_vmem,
                             sem).wait()

        pltpu.emit_pipeline(
            body,
            grid=(m // w,),
            in_specs=[pl.BlockSpec((1, 1, w), lambda i: (i, 0, 0))],
            out_specs=[pl.BlockSpec((w, H), lambda i: (i, 0))],
            core_axis_name=("core", "subcore"),
            dimension_semantics=(pltpu.PARALLEL,),
        )(idx_hbm, out_hbm)

    return k(table, idx3)


def sc_scatter_add(rows, idx, t):
    kn, m = idx.shape
    w = _win(m)
    idx3 = jnp.stack([idx[j].reshape(m // w, w) for j in range(kn)], axis=1)
    cz = 80
    nch = t // cz
    per_tile = -(-nch // 16)
    zrs = jnp.zeros((cz, H), jnp.float32)

    @functools.partial(
        pl.kernel,
        out_type=jax.ShapeDtypeStruct((2, t, H), jnp.float32),
        mesh=_SC_MESH,
        scratch_types=[pltpu.VMEM_SHARED((t, H), jnp.float32)],
    )
    def k(rows_hbm, idx_hbm, z_hbm, out_hbm, acc):
        cid = lax.axis_index("core")
        sid = lax.axis_index("subcore")

        @pl.loop(0, per_tile)
        def _(i):
            ch = i * 16 + sid

            @pl.when(ch < nch)
            def _():
                off = pl.multiple_of(ch * cz, 8)
                pltpu.sync_copy(z_hbm, acc.at[pl.ds(off, cz)])

        plsc.subcore_barrier()

        def body(rows_vmem, idx_vmem):
            for j in range(kn):
                pltpu.sync_copy(rows_vmem, acc.at[idx_vmem.at[0, j]],
                                add=True)

        pltpu.emit_pipeline(
            body,
            grid=(m // w,),
            in_specs=[pl.BlockSpec((w, H), lambda i: (i, 0)),
                      pl.BlockSpec((1, kn, w), lambda i: (i, 0, 0))],
            out_specs=[],
            core_axis_name=("core", "subcore"),
            dimension_semantics=(pltpu.PARALLEL,),
        )(rows_hbm, idx_hbm)

        plsc.subcore_barrier()

        @pl.loop(0, per_tile)
        def _(i):
            ch = i * 16 + sid

            @pl.when(ch < nch)
            def _():
                off = pl.multiple_of(ch * cz, 8)
                pltpu.sync_copy(acc.at[pl.ds(off, cz)],
                                out_hbm.at[cid].at[pl.ds(off, cz)])

    return k(rows, idx3, zrs)


def sc_segment_sum_c(rows, idx, c):
    m = idx.shape[0]
    w = 80
    assert m % w == 0 and c % 2 == 0
    half = c // 2
    pad = half + 80
    idx3 = idx.reshape(m // w, 1, w)
    cz = 80
    nzch = pad // cz
    noch = half // cz
    zrs = jnp.zeros((cz, H), jnp.float32)

    @functools.partial(
        pl.kernel,
        out_type=jax.ShapeDtypeStruct((2, half, H), jnp.float32),
        mesh=_SC_MESH,
        scratch_types=[pltpu.VMEM_SHARED((pad, H), jnp.float32),
                       pltpu.VMEM((1, w), jnp.int32)],
    )
    def k(rows_hbm, idx_hbm, z_hbm, out_hbm, acc, sidx):
        cid = lax.axis_index("core")
        sid = lax.axis_index("subcore")
        lo = cid * half

        @pl.loop(0, -(-nzch // 16))
        def _(i):
            ch = i * 16 + sid

            @pl.when(ch < nzch)
            def _():
                off = pl.multiple_of(ch * cz, 8)
                pltpu.sync_copy(z_hbm, acc.at[pl.ds(off, cz)])

        plsc.subcore_barrier()

        def body(rows_vmem, idx_vmem):
            for kk in range(w // 16):
                v = idx_vmem[0, 0, pl.ds(kk * 16, 16)]
                ok = (v >= lo) & (v < lo + half)
                sidx[0, pl.ds(kk * 16, 16)] = jnp.where(ok, v - lo, half)
            pltpu.sync_copy(rows_vmem, acc.at[sidx.at[0]], add=True)

        pltpu.emit_pipeline(
            body,
            grid=(m // w,),
            in_specs=[pl.BlockSpec((w, H), lambda i: (i, 0)),
                      pl.BlockSpec((1, 1, w), lambda i: (i, 0, 0))],
            out_specs=[],
            core_axis_name="subcore",
            dimension_semantics=(pltpu.PARALLEL,),
        )(rows_hbm, idx_hbm)

        plsc.subcore_barrier()

        @pl.loop(0, -(-noch // 16))
        def _(i):
            ch = i * 16 + sid

            @pl.when(ch < noch)
            def _():
                off = pl.multiple_of(ch * cz, 8)
                pltpu.sync_copy(acc.at[pl.ds(off, cz)],
                                out_hbm.at[cid].at[pl.ds(off, cz)])

    return k(rows, idx3, zrs).reshape(c, H)


def _mlp(parts, groups, p):
    u, su = matmul_stats(parts, p["W1"], groups)
    v, sv = bn_matmul_stats(u, su, p["g1"], p["b1"], p["W2"])
    return bn_act(v, sv, p["g2"], p["b2"])


def kernel(node_rep, edge_rep, cycle_rep, edge_index, cycle_node_ids,
           cycle_ids, node_mlp, edge_mlp0, cycle_mlp, edge_mlpc, edge_mlpt):
    n = node_rep.shape[0]
    c = 16000

    n2e = sc_gather2(node_rep, edge_index)
    u0, su0 = matmul_stats([edge_rep, n2e], edge_mlp0["W1"], [[0], [1]])
    v0, sv0 = bn_matmul_stats(u0, su0, edge_mlp0["g1"], edge_mlp0["b1"],
                              edge_mlp0["W2"])
    edge_out0 = bn_act(v0, sv0, edge_mlp0["g2"], edge_mlp0["b2"])

    e_at_n = add_halves(sc_scatter_add(edge_rep, edge_index, n))
    per_row = sc_gather(e_at_n, cycle_node_ids)
    cyc_sum = sc_segment_sum_c(per_row, cycle_ids, c)
    gcyc = sc_gather(cyc_sum, cycle_ids)
    cycle_out = _mlp([cycle_rep, per_row, gcyc], [[0], [1], [2]], cycle_mlp)

    c_at_n = add_halves(sc_scatter_add(cycle_out,
                                       cycle_node_ids.reshape(1, -1), n))
    c2e = sc_gather2(c_at_n, edge_index)

    e2n = sc_scatter_add(edge_out0, edge_index, n)
    node_out = _mlp([node_rep, e2n], [[0], [1]], node_mlp)

    uc, suc = matmul_stats([edge_rep, c2e], edge_mlpc["W1"], [[0], [1]])
    vc, svc = bn_matmul_stats(uc, suc, edge_mlpc["g1"], edge_mlpc["b1"],
                              edge_mlpc["W2"])

    ut, sut = bn2_matmul_stats(v0, sv0, edge_mlp0["g2"], edge_mlp0["b2"],
                               vc, svc, edge_mlpc["g2"], edge_mlpc["b2"],
                               edge_mlpt["W1"])
    vt, svt = bn_matmul_stats(ut, sut, edge_mlpt["g1"], edge_mlpt["b1"],
                              edge_mlpt["W2"])
    edge_out = bn_act(vt, svt, edge_mlpt["g2"], edge_mlpt["b2"])

    return (node_out, edge_out, cycle_out)


# speedup vs baseline: 3.2603x; 1.0011x over previous
import functools

import jax
import jax.numpy as jnp
from jax import lax
from jax.experimental import pallas as pl
from jax.experimental.pallas import tpu as pltpu

H = 128
EPS = 1e-5


def _pick_block(m, cap=8000):
    for b in (16000, 8000, 4000, 2000, 1000, 500, 250, 125, 100, 40, 8):
        if b <= cap and m % b == 0:
            return b
    return m


def _bf(x):
    return x


def _k1_body(groups, nparts, *refs):
    part_refs = refs[:nparts]
    w_ref = refs[nparts]
    u_ref, s_ref = refs[nparts + 1], refs[nparts + 2]
    acc = None
    for gi, members in enumerate(groups):
        xg = None
        for pix in members:
            r = part_refs[pix]
            if len(r.shape) == 3:
                xv = r[0] + r[1]
            else:
                xv = r[...]
            xg = xv if xg is None else xg + xv
        wg = w_ref[pl.ds(gi * H, H), :]
        p = jax.lax.dot_general(_bf(xg), _bf(wg), (((1,), (0,)), ((), ())),
                                preferred_element_type=jnp.float32)
        acc = p if acc is None else acc + p
    u_ref[...] = acc
    s1 = jnp.sum(acc, axis=0, keepdims=True)
    s2 = jnp.sum(acc * acc, axis=0, keepdims=True)
    st = jnp.concatenate([s1, s2], axis=0)

    @pl.when(pl.program_id(0) == 0)
    def _():
        s_ref[...] = st

    @pl.when(pl.program_id(0) != 0)
    def _():
        s_ref[...] = s_ref[...] + st


def matmul_stats(parts, w1, groups):
    m = parts[0].shape[-2]
    b = _pick_block(m)
    dout = w1.shape[1]
    in_specs = []
    for p in parts:
        if p.ndim == 3:
            in_specs.append(pl.BlockSpec((2, b, H), lambda i: (0, i, 0)))
        else:
            in_specs.append(pl.BlockSpec((b, H), lambda i: (i, 0)))
    in_specs.append(pl.BlockSpec(w1.shape, lambda i: (0, 0)))
    out_shape = [jax.ShapeDtypeStruct((m, dout), jnp.float32),
                 jax.ShapeDtypeStruct((2, dout), jnp.float32)]
    out_specs = [pl.BlockSpec((b, dout), lambda i: (i, 0)),
                 pl.BlockSpec((2, dout), lambda i: (0, 0))]
    return pl.pallas_call(
        functools.partial(_k1_body, groups, len(parts)),
        grid=(m // b,),
        in_specs=in_specs,
        out_specs=out_specs,
        out_shape=out_shape,
    )(*parts, w1)


def _bn_coeffs(s_ref, g_ref, b_ref, inv_m):
    mean = s_ref[0:1, :] * inv_m
    var = s_ref[1:2, :] * inv_m - mean * mean
    scale = g_ref[...] * jax.lax.rsqrt(var + EPS)
    shift = b_ref[...] - mean * scale
    return scale, shift


def _k2_body(inv_m, u_ref, s_ref, g_ref, b_ref, w_ref, v_ref, sv_ref):
    scale, shift = _bn_coeffs(s_ref, g_ref, b_ref, inv_m)
    h = jnp.maximum(u_ref[...] * scale + shift, 0.0)
    v = jax.lax.dot_general(_bf(h), _bf(w_ref[...]), (((1,), (0,)), ((), ())),
                            preferred_element_type=jnp.float32)
    v_ref[...] = v
    s1 = jnp.sum(v, axis=0, keepdims=True)
    s2 = jnp.sum(v * v, axis=0, keepdims=True)
    st = jnp.concatenate([s1, s2], axis=0)

    @pl.when(pl.program_id(0) == 0)
    def _():
        sv_ref[...] = st

    @pl.when(pl.program_id(0) != 0)
    def _():
        sv_ref[...] = sv_ref[...] + st


def bn_matmul_stats(u, s_u, g, bb, w2):
    m, din = u.shape
    dout = w2.shape[1]
    b = _pick_block(m)
    return pl.pallas_call(
        functools.partial(_k2_body, 1.0 / m),
        grid=(m // b,),
        in_specs=[pl.BlockSpec((b, din), lambda i: (i, 0)),
                  pl.BlockSpec((2, din), lambda i: (0, 0)),
                  pl.BlockSpec((1, din), lambda i: (0, 0)),
                  pl.BlockSpec((1, din), lambda i: (0, 0)),
                  pl.BlockSpec((din, dout), lambda i: (0, 0))],
        out_specs=[pl.BlockSpec((b, dout), lambda i: (i, 0)),
                   pl.BlockSpec((2, dout), lambda i: (0, 0))],
        out_shape=[jax.ShapeDtypeStruct((m, dout), jnp.float32),
                   jax.ShapeDtypeStruct((2, dout), jnp.float32)],
    )(u, s_u, g.reshape(1, din), bb.reshape(1, din), w2)


def _k3_body(inv_m, v_ref, s_ref, g_ref, b_ref, y_ref):
    scale, shift = _bn_coeffs(s_ref, g_ref, b_ref, inv_m)
    y_ref[...] = jnp.maximum(v_ref[...] * scale + shift, 0.0)


def bn_act(v, s_v, g, bb):
    m, d = v.shape
    b = _pick_block(m, 16000)
    return pl.pallas_call(
        functools.partial(_k3_body, 1.0 / m),
        grid=(m // b,),
        in_specs=[pl.BlockSpec((b, d), lambda i: (i, 0)),
                  pl.BlockSpec((2, d), lambda i: (0, 0)),
                  pl.BlockSpec((1, d), lambda i: (0, 0)),
                  pl.BlockSpec((1, d), lambda i: (0, 0))],
        out_specs=pl.BlockSpec((b, d), lambda i: (i, 0)),
        out_shape=jax.ShapeDtypeStruct((m, d), jnp.float32),
    )(v, s_v, g.reshape(1, d), bb.reshape(1, d))


def _k1f_body(inv_m, v0_ref, s0_ref, g0_ref, b0_ref,
              v2_ref, s2_ref, g2_ref, b2_ref, w_ref, u_ref, s_ref):
    sc0, sh0 = _bn_coeffs(s0_ref, g0_ref, b0_ref, inv_m)
    sc2, sh2 = _bn_coeffs(s2_ref, g2_ref, b2_ref, inv_m)
    x0 = jnp.maximum(v0_ref[...] * sc0 + sh0, 0.0)
    x2 = jnp.maximum(v2_ref[...] * sc2 + sh2, 0.0)
    u = (jax.lax.dot_general(_bf(x0), _bf(w_ref[pl.ds(0, H), :]),
                             (((1,), (0,)), ((), ())),
                             preferred_element_type=jnp.float32)
         + jax.lax.dot_general(_bf(x2), _bf(w_ref[pl.ds(H, H), :]),
                               (((1,), (0,)), ((), ())),
                               preferred_element_type=jnp.float32))
    u_ref[...] = u
    s1 = jnp.sum(u, axis=0, keepdims=True)
    s2 = jnp.sum(u * u, axis=0, keepdims=True)
    st = jnp.concatenate([s1, s2], axis=0)

    @pl.when(pl.program_id(0) == 0)
    def _():
        s_ref[...] = st

    @pl.when(pl.program_id(0) != 0)
    def _():
        s_ref[...] = s_ref[...] + st


def bn2_matmul_stats(v0, s0, g0, b0, v2, s2, g2, b2, w1):
    m = v0.shape[0]
    b = _pick_block(m)
    dout = w1.shape[1]
    sm = pl.BlockSpec((2, H), lambda i: (0, 0))
    gm = pl.BlockSpec((1, H), lambda i: (0, 0))
    return pl.pallas_call(
        functools.partial(_k1f_body, 1.0 / m),
        grid=(m // b,),
        in_specs=[pl.BlockSpec((b, H), lambda i: (i, 0)), sm, gm, gm,
                  pl.BlockSpec((b, H), lambda i: (i, 0)), sm, gm, gm,
                  pl.BlockSpec((2 * H, dout), lambda i: (0, 0))],
        out_specs=[pl.BlockSpec((b, dout), lambda i: (i, 0)),
                   pl.BlockSpec((2, dout), lambda i: (0, 0))],
        out_shape=[jax.ShapeDtypeStruct((m, dout), jnp.float32),
                   jax.ShapeDtypeStruct((2, dout), jnp.float32)],
    )(v0, s0, g0.reshape(1, H), b0.reshape(1, H),
      v2, s2, g2.reshape(1, H), b2.reshape(1, H), w1)


def _add2_body(x_ref, o_ref):
    o_ref[...] = x_ref[0] + x_ref[1]


def add_halves(x2):
    _, t, d = x2.shape
    b = _pick_block(t)
    return pl.pallas_call(
        _add2_body,
        grid=(t // b,),
        in_specs=[pl.BlockSpec((2, b, d), lambda i: (0, i, 0))],
        out_specs=pl.BlockSpec((b, d), lambda i: (i, 0)),
        out_shape=jax.ShapeDtypeStruct((t, d), jnp.float32),
    )(x2)


from jax.experimental.pallas import tpu_sc as plsc

_SC_MESH = plsc.VectorSubcoreMesh(core_axis_name="core",
                                  subcore_axis_name="subcore")


def _win(m, cap=128):
    for w in (128, 88, 80, 64, 40, 16):
        if w <= cap and m % w == 0:
            return w
    raise ValueError(m)


def sc_gather2(table, idx2):
    m = idx2.shape[1]
    w = _win(m)
    idx3 = jnp.stack([idx2[0].reshape(m // w, w), idx2[1].reshape(m // w, w)],
                     axis=1)

    @functools.partial(
        pl.kernel,
        out_type=jax.ShapeDtypeStruct((2, m, H), jnp.float32),
        mesh=_SC_MESH,
        scratch_types=[pltpu.SemaphoreType.DMA, pltpu.SemaphoreType.DMA],
    )
    def k(table_hbm, idx_hbm, out_hbm, sem0, sem1):
        def body(idx_vmem, out_vmem):
            c0 = pltpu.async_copy(table_hbm.at[idx_vmem.at[0, 0]],
                                  out_vmem.at[0], sem0)
            c1 = pltpu.async_copy(table_hbm.at[idx_vmem.at[0, 1]],
                                  out_vmem.at[1], sem1)
            c0.wait()
            c1.wait()

        pltpu.emit_pipeline(
            body,
            grid=(m // w,),
            in_specs=[pl.BlockSpec((1, 2, w), lambda i: (i, 0, 0))],
            out_specs=[pl.BlockSpec((2, w, H), lambda i: (0, i, 0))],
            core_axis_name=("core", "subcore"),
            dimension_semantics=(pltpu.PARALLEL,),
        )(idx_hbm, out_hbm)

    return k(table, idx3)


def sc_gather(table, idx):
    m = idx.shape[0]
    w = _win(m)
    idx3 = idx.reshape(m // w, 1, w)

    @functools.partial(
        pl.kernel,
        out_type=jax.ShapeDtypeStruct((m, H), jnp.float32),
        mesh=_SC_MESH,
        scratch_types=[pltpu.SemaphoreType.DMA],
    )
    def k(table_hbm, idx_hbm, out_hbm, sem):
        def body(idx_vmem, out_vmem):
            pltpu.async_copy(table_hbm.at[idx_vmem.at[0, 0]], out_vmem,
                             sem).wait()

        pltpu.emit_pipeline(
            body,
            grid=(m // w,),
            in_specs=[pl.BlockSpec((1, 1, w), lambda i: (i, 0, 0))],
            out_specs=[pl.BlockSpec((w, H), lambda i: (i, 0))],
            core_axis_name=("core", "subcore"),
            dimension_semantics=(pltpu.PARALLEL,),
        )(idx_hbm, out_hbm)

    return k(table, idx3)


def sc_scatter_add(rows, idx, t):
    kn, m = idx.shape
    w = _win(m)
    idx3 = jnp.stack([idx[j].reshape(m // w, w) for j in range(kn)], axis=1)
    cz = 80
    nch = t // cz
    per_tile = -(-nch // 16)
    zrs = jnp.zeros((cz, H), jnp.float32)

    @functools.partial(
        pl.kernel,
        out_type=jax.ShapeDtypeStruct((2, t, H), jnp.float32),
        mesh=_SC_MESH,
        scratch_types=[pltpu.VMEM_SHARED((t, H), jnp.float32)],
    )
    def k(rows_hbm, idx_hbm, z_hbm, out_hbm, acc):
        cid = lax.axis_index("core")
        sid = lax.axis_index("subcore")

        @pl.loop(0, per_tile)
        def _(i):
            ch = i * 16 + sid

            @pl.when(ch < nch)
            def _():
                off = pl.multiple_of(ch * cz, 8)
                pltpu.sync_copy(z_hbm, acc.at[pl.ds(off, cz)])

        plsc.subcore_barrier()

        def body(rows_vmem, idx_vmem):
            for j in range(kn):
                pltpu.sync_copy(rows_vmem, acc.at[idx_vmem.at[0, j]],
                                add=True)

        pltpu.emit_pipeline(
            body,
            grid=(m // w,),
            in_specs=[pl.BlockSpec((w, H), lambda i: (i, 0)),
                      pl.BlockSpec((1, kn, w), lambda i: (i, 0, 0))],
            out_specs=[],
            core_axis_name=("core", "subcore"),
            dimension_semantics=(pltpu.PARALLEL,),
        )(rows_hbm, idx_hbm)

        plsc.subcore_barrier()

        @pl.loop(0, per_tile)
        def _(i):
            ch = i * 16 + sid

            @pl.when(ch < nch)
            def _():
                off = pl.multiple_of(ch * cz, 8)
                pltpu.sync_copy(acc.at[pl.ds(off, cz)],
                                out_hbm.at[cid].at[pl.ds(off, cz)])

    return k(rows, idx3, zrs)


def sc_segment_sum_c(rows, idx, c):
    m = idx.shape[0]
    w = 80
    assert m % w == 0 and c % 2 == 0
    half = c // 2
    pad = half + 80
    idx3 = idx.reshape(m // w, 1, w)
    cz = 80
    nzch = pad // cz
    noch = half // cz
    zrs = jnp.zeros((cz, H), jnp.float32)

    @functools.partial(
        pl.kernel,
        out_type=jax.ShapeDtypeStruct((2, half, H), jnp.float32),
        mesh=_SC_MESH,
        scratch_types=[pltpu.VMEM_SHARED((pad, H), jnp.float32),
                       pltpu.VMEM((1, w), jnp.int32)],
    )
    def k(rows_hbm, idx_hbm, z_hbm, out_hbm, acc, sidx):
        cid = lax.axis_index("core")
        sid = lax.axis_index("subcore")
        lo = cid * half

        @pl.loop(0, -(-nzch // 16))
        def _(i):
            ch = i * 16 + sid

            @pl.when(ch < nzch)
            def _():
                off = pl.multiple_of(ch * cz, 8)
                pltpu.sync_copy(z_hbm, acc.at[pl.ds(off, cz)])

        plsc.subcore_barrier()

        def body(rows_vmem, idx_vmem):
            for kk in range(w // 16):
                v = idx_vmem[0, 0, pl.ds(kk * 16, 16)]
                ok = (v >= lo) & (v < lo + half)
                sidx[0, pl.ds(kk * 16, 16)] = jnp.where(ok, v - lo, half)
            pltpu.sync_copy(rows_vmem, acc.at[sidx.at[0]], add=True)

        pltpu.emit_pipeline(
            body,
            grid=(m // w,),
            in_specs=[pl.BlockSpec((w, H), lambda i: (i, 0)),
                      pl.BlockSpec((1, 1, w), lambda i: (i, 0, 0))],
            out_specs=[],
            core_axis_name="subcore",
            dimension_semantics=(pltpu.PARALLEL,),
        )(rows_hbm, idx_hbm)

        plsc.subcore_barrier()

        @pl.loop(0, -(-noch // 16))
        def _(i):
            ch = i * 16 + sid

            @pl.when(ch < noch)
            def _():
                off = pl.multiple_of(ch * cz, 8)
                pltpu.sync_copy(acc.at[pl.ds(off, cz)],
                                out_hbm.at[cid].at[pl.ds(off, cz)])

    return k(rows, idx3, zrs).reshape(c, H)


def _mlp(parts, groups, p):
    u, su = matmul_stats(parts, p["W1"], groups)
    v, sv = bn_matmul_stats(u, su, p["g1"], p["b1"], p["W2"])
    return bn_act(v, sv, p["g2"], p["b2"])


def kernel(node_rep, edge_rep, cycle_rep, edge_index, cycle_node_ids,
           cycle_ids, node_mlp, edge_mlp0, cycle_mlp, edge_mlpc, edge_mlpt):
    n = node_rep.shape[0]
    c = 16000

    n2e = sc_gather2(node_rep, edge_index)
    u0, su0 = matmul_stats([edge_rep, n2e], edge_mlp0["W1"], [[0], [1]])
    v0, sv0 = bn_matmul_stats(u0, su0, edge_mlp0["g1"], edge_mlp0["b1"],
                              edge_mlp0["W2"])
    edge_out0 = bn_act(v0, sv0, edge_mlp0["g2"], edge_mlp0["b2"])

    e_at_n = add_halves(sc_scatter_add(edge_rep, edge_index, n))
    per_row = sc_gather(e_at_n, cycle_node_ids)
    cyc_sum = sc_segment_sum_c(per_row, cycle_ids, c)
    gcyc = sc_gather(cyc_sum, cycle_ids)
    cycle_out = _mlp([cycle_rep, per_row, gcyc], [[0], [1], [2]], cycle_mlp)

    c_at_n = add_halves(sc_scatter_add(cycle_out,
                                       cycle_node_ids.reshape(1, -1), n))
    c2e = sc_gather2(c_at_n, edge_index)

    e2n = sc_scatter_add(edge_out0, edge_index, n)
    node_out = _mlp([node_rep, e2n], [[0], [1]], node_mlp)

    uc, suc = matmul_stats([edge_rep, c2e], edge_mlpc["W1"], [[0], [1]])
    vc, svc = bn_matmul_stats(uc, suc, edge_mlpc["g1"], edge_mlpc["b1"],
                              edge_mlpc["W2"])

    ut, sut = bn2_matmul_stats(v0, sv0, edge_mlp0["g2"], edge_mlp0["b2"],
                               vc, svc, edge_mlpc["g2"], edge_mlpc["b2"],
                               edge_mlpt["W1"])
    vt, svt = bn_matmul_stats(ut, sut, edge_mlpt["g1"], edge_mlpt["b1"],
                              edge_mlpt["W2"])
    edge_out = bn_act(vt, svt, edge_mlpt["g2"], edge_mlpt["b2"])

    return (node_out, edge_out, cycle_out)
